# Initial kernel scaffold; baseline (speedup 1.0000x reference)
import jax
import jax.numpy as jnp
from jax.experimental import pallas as pl


def kernel(x_tx, x_addr, ei_tx_tx, ei_addr_addr, ei_addr_tx, ei_tx_addr, l1_tt_W, l1_tt_b, l1_aa_W, l1_aa_b, l1_at_Wl, l1_at_bl, l1_at_Wr, l1_ta_Wl, l1_ta_bl, l1_ta_Wr, l2_tt_W, l2_tt_b, l2_aa_W, l2_aa_b, l2_at_Wl, l2_at_bl, l2_at_Wr, l2_ta_Wl, l2_ta_bl, l2_ta_Wr):
    raise NotImplementedError("write your pallas kernel here")


# speedup vs baseline: 3.6880x; 3.6880x over previous
import functools

import jax
import jax.numpy as jnp
from jax import lax
from jax.experimental import pallas as pl
from jax.experimental.pallas import tpu as pltpu
from jax.experimental.pallas import tpu_sc as plsc

N = 50000
E = 400000
D_IN = 128
D_H = 128
D_OUT = 64

NC = 2
NS = 16
CW = 16

EB = 128
ROWS = 196
EPT_PAD = ROWS * EB
E_PAD = NS * EPT_PAD
DUMP = N
ACC_ROWS = 50176
ZPT = ACC_ROWS // NS
ZROWS = 112

C1 = D_H // CW
C2 = D_OUT // CW
R = 1000
GRID = N // R

_MESH = dict(core_axis_name="c", subcore_axis_name="s")


def _hist_body(d_tt, d_aa, d_at, d_ta, hist_out, ones_v, zbuf, fbuf, dbuf, acc):
    core = lax.axis_index("c")
    s = lax.axis_index("s")
    d_refs = (d_tt, d_aa, d_at, d_ta)

    def fill(i, _):
        ones_v[pl.ds(i * 16, 16)] = jnp.ones((16,), jnp.float32)
        zbuf[pl.ds(i * 16, 16)] = jnp.zeros((16,), jnp.float32)
        return _

    lax.fori_loop(0, ZPT // 16, fill, None)

    def do_rel(rel):
        pltpu.sync_copy(zbuf, acc.at[pl.ds(s * ZPT, ZPT)])
        plsc.subcore_barrier()
        pltpu.sync_copy(d_refs[rel].at[s], dbuf)

        def step(j, _):
            pltpu.sync_copy(ones_v.at[pl.ds(0, EB)], acc.at[dbuf.at[j]],
                            add=True)
            return _

        lax.fori_loop(0, ROWS, step, None)
        plsc.subcore_barrier()
        pltpu.sync_copy(acc.at[pl.ds(s * ZPT, ZPT)], fbuf)
        pltpu.sync_copy(fbuf, hist_out.at[pl.ds(rel * ACC_ROWS + s * ZPT,
                                                ZPT)])
        plsc.subcore_barrier()

    for c_py in range(NC):
        @pl.when(core == c_py)
        def _():
            for rel in (c_py, c_py + 2):
                do_rel(rel)


def _hist_kernel(d_tt, d_aa, d_at, d_ta):
    k = pl.kernel(
        _hist_body,
        out_type=jax.ShapeDtypeStruct((4 * ACC_ROWS,), jnp.float32),
        mesh=plsc.VectorSubcoreMesh(**_MESH),
        scratch_types=[
            pltpu.VMEM((ZPT,), jnp.float32),
            pltpu.VMEM((ZPT,), jnp.float32),
            pltpu.VMEM((ZPT,), jnp.float32),
            pltpu.VMEM((ROWS, EB), jnp.int32),
            pltpu.VMEM_SHARED((ACC_ROWS,), jnp.float32),
        ],
    )
    return k(d_tt, d_aa, d_at, d_ta)


def _scatter_body(n_chunks, t_tt, t_at, t_aa, t_ta,
                  s_tt, d_tt, s_at, d_at, s_aa, d_aa, s_ta, d_ta,
                  o_tt, o_at, o_aa, o_ta,
                  sbuf, dbuf, gbuf, zbuf, fbuf, acc):
    cps = n_chunks // NC
    core = lax.axis_index("c")
    s = lax.axis_index("s")
    tabs = (t_tt, t_at, t_aa, t_ta)
    srcs = (s_tt, s_at, s_aa, s_ta)
    dsts = (d_tt, d_at, d_aa, d_ta)
    outs = (o_tt, o_at, o_aa, o_ta)

    def fill(i, _):
        zbuf[i % ZROWS, pl.ds((i // ZROWS) * 16, 16)] = jnp.zeros(
            (16,), jnp.float32)
        return _

    lax.fori_loop(0, ZROWS * (CW // 16), fill, None)

    for rel in range(4):
        pltpu.sync_copy(srcs[rel].at[s], sbuf)
        pltpu.sync_copy(dsts[rel].at[s], dbuf)
        for ci in range(cps):
            off = jnp.where(ci == 0, core * cps * N, N).astype(jnp.int32)

            def offset(j, _):
                for k in range(EB // 16):
                    sl = pl.ds(k * 16, 16)
                    sbuf[j, sl] = sbuf[j, sl] + off
                return _

            lax.fori_loop(0, ROWS, offset, None)
            for z in range(ZPT // ZROWS):
                pltpu.sync_copy(zbuf, acc.at[pl.ds(s * ZPT + z * ZROWS,
                                                   ZROWS)])
            plsc.subcore_barrier()

            def edge(j, _):
                pltpu.sync_copy(tabs[rel].at[sbuf.at[j]], gbuf)
                pltpu.sync_copy(gbuf, acc.at[dbuf.at[j]], add=True)
                return _

            lax.fori_loop(0, ROWS, edge, None)
            plsc.subcore_barrier()
            obase = (core * cps + ci) * ACC_ROWS
            for z in range(ZPT // ZROWS):
                sl = pl.ds(s * ZPT + z * ZROWS, ZROWS)
                pltpu.sync_copy(acc.at[sl], fbuf)
                pltpu.sync_copy(fbuf, outs[rel].at[pl.ds(
                    obase + s * ZPT + z * ZROWS, ZROWS)])
            plsc.subcore_barrier()


def _sc_scatter(n_chunks, tabs, edges):
    out_t = tuple(jax.ShapeDtypeStruct((n_chunks * ACC_ROWS, CW), jnp.float32)
                  for _ in range(4))
    k = pl.kernel(
        functools.partial(_scatter_body, n_chunks),
        out_type=out_t,
        mesh=plsc.VectorSubcoreMesh(**_MESH),
        scratch_types=[
            pltpu.VMEM((ROWS, EB), jnp.int32),
            pltpu.VMEM((ROWS, EB), jnp.int32),
            pltpu.VMEM((EB, CW), jnp.float32),
            pltpu.VMEM((ZROWS, CW), jnp.float32),
            pltpu.VMEM((ZROWS, CW), jnp.float32),
            pltpu.VMEM_SHARED((ACC_ROWS, CW), jnp.float32),
        ],
        compiler_params=pltpu.CompilerParams(use_tc_tiling_on_sc=False),
    )
    return k(*tabs, *edges)


def _mm1_body(x_ref, wg_ref, wl_ref, wr_ref, hist_ref, g_ref, p_ref, r_ref):
    x = x_ref[...]
    dinv = lax.rsqrt(hist_ref[...] + 1.0)
    g = jnp.dot(x, wg_ref[...], preferred_element_type=jnp.float32) * dinv
    p = jnp.dot(x, wl_ref[...], preferred_element_type=jnp.float32)
    r = jnp.dot(x, wr_ref[...], preferred_element_type=jnp.float32)
    for c in range(C1):
        sl = slice(c * CW, (c + 1) * CW)
        g_ref[c, :, :] = g[:, sl]
        p_ref[c, :, :] = p[:, sl]
        r_ref[c, :, :] = r[:, sl]


def _mm1(x, wg, wl, wr, hist):
    chunked = pl.BlockSpec((C1, R, CW), lambda i: (0, i, 0))
    return pl.pallas_call(
        _mm1_body,
        grid=(GRID,),
        in_specs=[
            pl.BlockSpec((R, D_IN), lambda i: (i, 0)),
            pl.BlockSpec((D_IN, D_H), lambda i: (0, 0)),
            pl.BlockSpec((D_IN, D_H), lambda i: (0, 0)),
            pl.BlockSpec((D_IN, D_H), lambda i: (0, 0)),
            pl.BlockSpec((R, 1), lambda i: (i, 0)),
        ],
        out_specs=[chunked, chunked, chunked],
        out_shape=[jax.ShapeDtypeStruct((C1, N, CW), jnp.float32)] * 3,
    )(x, wg, wl, wr, hist)


def _mm2_body(s_ref, g_ref, a_ref, rr_ref, hist_ref, cnt_ref, b_ref,
              wg_ref, wl_ref, wr_ref, g2_ref, p2_ref, r2_ref):
    dinv = lax.rsqrt(hist_ref[...] + 1.0)
    cinv = 1.0 / jnp.maximum(cnt_ref[...], 1.0)
    h2 = jnp.zeros((R, D_OUT), jnp.float32)
    p2 = jnp.zeros((R, D_OUT), jnp.float32)
    r2 = jnp.zeros((R, D_OUT), jnp.float32)
    for c in range(C1):
        piece = jnp.maximum(
            dinv * (s_ref[c, :, :] + g_ref[c, :, :])
            + a_ref[c, :, :] * cinv + rr_ref[c, :, :] + b_ref[c], 0.0)
        sl = slice(c * CW, (c + 1) * CW)
        h2 += jnp.dot(piece, wg_ref[sl, :], preferred_element_type=jnp.float32)
        p2 += jnp.dot(piece, wl_ref[sl, :], preferred_element_type=jnp.float32)
        r2 += jnp.dot(piece, wr_ref[sl, :], preferred_element_type=jnp.float32)
    g2 = dinv * h2
    for c in range(C2):
        sl = slice(c * CW, (c + 1) * CW)
        g2_ref[c, :, :] = g2[:, sl]
        p2_ref[c, :, :] = p2[:, sl]
    r2_ref[...] = r2


def _mm2(s, g, a, rr, hist, cnt, b, wg, wl, wr):
    in4 = pl.BlockSpec((C1, R, CW), lambda i: (0, i, 0))
    col = pl.BlockSpec((R, 1), lambda i: (i, 0))
    w = pl.BlockSpec((D_H, D_OUT), lambda i: (0, 0))
    out2 = pl.BlockSpec((C2, R, CW), lambda i: (0, i, 0))
    return pl.pallas_call(
        _mm2_body,
        grid=(GRID,),
        in_specs=[in4, in4, in4, in4, col, col,
                  pl.BlockSpec((C1, CW), lambda i: (0, 0)), w, w, w],
        out_specs=[out2, out2, pl.BlockSpec((R, D_OUT), lambda i: (i, 0))],
        out_shape=[
            jax.ShapeDtypeStruct((C2, N, CW), jnp.float32),
            jax.ShapeDtypeStruct((C2, N, CW), jnp.float32),
            jax.ShapeDtypeStruct((N, D_OUT), jnp.float32),
        ],
    )(s, g, a, rr, hist, cnt, b, wg, wl, wr)


def _fin_body(s_ref, g_ref, a_ref, r2_ref, hist_ref, cnt_ref, b_ref, out_ref):
    dinv = lax.rsqrt(hist_ref[...] + 1.0)
    cinv = 1.0 / jnp.maximum(cnt_ref[...], 1.0)
    for c in range(C2):
        sl = slice(c * CW, (c + 1) * CW)
        out_ref[:, sl] = (dinv * (s_ref[c, :, :] + g_ref[c, :, :])
                          + a_ref[c, :, :] * cinv + b_ref[c]
                          + r2_ref[:, sl])


def _fin(s, g, a, r2, hist, cnt, b):
    in2 = pl.BlockSpec((C2, R, CW), lambda i: (0, i, 0))
    col = pl.BlockSpec((R, 1), lambda i: (i, 0))
    return pl.pallas_call(
        _fin_body,
        grid=(GRID,),
        in_specs=[in2, in2, in2,
                  pl.BlockSpec((R, D_OUT), lambda i: (i, 0)), col, col,
                  pl.BlockSpec((C2, CW), lambda i: (0, 0))],
        out_specs=pl.BlockSpec((R, D_OUT), lambda i: (i, 0)),
        out_shape=jax.ShapeDtypeStruct((N, D_OUT), jnp.float32),
    )(s, g, a, r2, hist, cnt, b)


def _prep_edges(ei):
    pad = E_PAD - E
    src = jnp.concatenate([ei[0], jnp.zeros((pad,), jnp.int32)])
    dst = jnp.concatenate([ei[1], jnp.full((pad,), DUMP, jnp.int32)])
    return src.reshape(NS, ROWS, EB), dst.reshape(NS, ROWS, EB)


def kernel(x_tx, x_addr, ei_tx_tx, ei_addr_addr, ei_addr_tx, ei_tx_addr,
           l1_tt_W, l1_tt_b, l1_aa_W, l1_aa_b,
           l1_at_Wl, l1_at_bl, l1_at_Wr,
           l1_ta_Wl, l1_ta_bl, l1_ta_Wr,
           l2_tt_W, l2_tt_b, l2_aa_W, l2_aa_b,
           l2_at_Wl, l2_at_bl, l2_at_Wr,
           l2_ta_Wl, l2_ta_bl, l2_ta_Wr):
    s_tt, d_tt = _prep_edges(ei_tx_tx)
    s_aa, d_aa = _prep_edges(ei_addr_addr)
    s_at, d_at = _prep_edges(ei_addr_tx)
    s_ta, d_ta = _prep_edges(ei_tx_addr)
    edges = (s_tt, d_tt, s_at, d_at, s_aa, d_aa, s_ta, d_ta)

    hist = _hist_kernel(d_tt, d_aa, d_at, d_ta).reshape(4, ACC_ROWS)
    h_tt = hist[0, :N].reshape(N, 1)
    h_aa = hist[1, :N].reshape(N, 1)
    c_at = hist[2, :N].reshape(N, 1)
    c_ta = hist[3, :N].reshape(N, 1)

    g_t, p_t, r_t = _mm1(x_tx, l1_tt_W, l1_ta_Wl, l1_at_Wr, h_tt)
    g_a, p_a, r_a = _mm1(x_addr, l1_aa_W, l1_at_Wl, l1_ta_Wr, h_aa)

    tabs1 = (g_t.reshape(C1 * N, CW), p_a.reshape(C1 * N, CW),
             g_a.reshape(C1 * N, CW), p_t.reshape(C1 * N, CW))
    S_tt, A_at, S_aa, A_ta = _sc_scatter(C1, tabs1, edges)

    b1_t = (l1_tt_b + l1_at_bl).reshape(C1, CW)
    b1_a = (l1_aa_b + l1_ta_bl).reshape(C1, CW)
    g2_t, p2_t, r2_t = _mm2(S_tt.reshape(C1, ACC_ROWS, CW), g_t,
                            A_at.reshape(C1, ACC_ROWS, CW), r_t, h_tt, c_at,
                            b1_t, l2_tt_W, l2_ta_Wl, l2_at_Wr)
    g2_a, p2_a, r2_a = _mm2(S_aa.reshape(C1, ACC_ROWS, CW), g_a,
                            A_ta.reshape(C1, ACC_ROWS, CW), r_a, h_aa, c_ta,
                            b1_a, l2_aa_W, l2_at_Wl, l2_ta_Wr)

    tabs2 = (g2_t.reshape(C2 * N, CW), p2_a.reshape(C2 * N, CW),
             g2_a.reshape(C2 * N, CW), p2_t.reshape(C2 * N, CW))
    S2_tt, A2_at, S2_aa, A2_ta = _sc_scatter(C2, tabs2, edges)

    b2_t = (l2_tt_b + l2_at_bl).reshape(C2, CW)
    b2_a = (l2_aa_b + l2_ta_bl).reshape(C2, CW)
    t2 = _fin(S2_tt.reshape(C2, ACC_ROWS, CW), g2_t,
              A2_at.reshape(C2, ACC_ROWS, CW), r2_t, h_tt, c_at, b2_t)
    a2 = _fin(S2_aa.reshape(C2, ACC_ROWS, CW), g2_a,
              A2_ta.reshape(C2, ACC_ROWS, CW), r2_a, h_aa, c_ta, b2_a)
    return (t2, a2)


# speedup vs baseline: 5.6103x; 1.5212x over previous
import functools

import jax
import jax.numpy as jnp
from jax import lax
from jax.experimental import pallas as pl
from jax.experimental.pallas import tpu as pltpu
from jax.experimental.pallas import tpu_sc as plsc

N = 50000
E = 400000
D_IN = 128
D_H = 128
D_OUT = 64

NC = 2
NS = 16
CW = 16

EB = 128
ROWS = 196
EPT_PAD = ROWS * EB
E_PAD = NS * EPT_PAD
DUMP = N
ACC_ROWS = 50176
ZPT = ACC_ROWS // NS
ZROWS = 112
NB = 7

C1 = D_H // CW
C2 = D_OUT // CW
R = 1000
GRID = N // R

_MESH = dict(core_axis_name="c", subcore_axis_name="s")


def _hist_body(d_tt, d_aa, d_at, d_ta, hist_out, ones_v, zbuf, fbuf, dbuf,
               acc, hsem):
    core = lax.axis_index("c")
    s = lax.axis_index("s")
    d_refs = (d_tt, d_aa, d_at, d_ta)

    def fill(i, _):
        ones_v[pl.ds(i * 16, 16)] = jnp.ones((16,), jnp.float32)
        zbuf[pl.ds(i * 16, 16)] = jnp.zeros((16,), jnp.float32)
        return _

    lax.fori_loop(0, ZPT // 16, fill, None)

    def do_rel(rel):
        pltpu.sync_copy(zbuf, acc.at[pl.ds(s * ZPT, ZPT)])
        plsc.subcore_barrier()
        pltpu.sync_copy(d_refs[rel].at[s], dbuf)

        def step(g, _):
            ds = [pltpu.async_copy(ones_v.at[pl.ds(0, EB)],
                                   acc.at[dbuf.at[g * NB + b]], hsem,
                                   add=True)
                  for b in range(NB)]
            for d in ds:
                d.wait()
            return _

        lax.fori_loop(0, ROWS // NB, step, None)
        plsc.subcore_barrier()
        pltpu.sync_copy(acc.at[pl.ds(s * ZPT, ZPT)], fbuf)
        pltpu.sync_copy(fbuf, hist_out.at[pl.ds(rel * ACC_ROWS + s * ZPT,
                                                ZPT)])
        plsc.subcore_barrier()

    for c_py in range(NC):
        @pl.when(core == c_py)
        def _():
            for rel in (c_py, c_py + 2):
                do_rel(rel)


def _hist_kernel(d_tt, d_aa, d_at, d_ta):
    k = pl.kernel(
        _hist_body,
        out_type=jax.ShapeDtypeStruct((4 * ACC_ROWS,), jnp.float32),
        mesh=plsc.VectorSubcoreMesh(**_MESH),
        scratch_types=[
            pltpu.VMEM((ZPT,), jnp.float32),
            pltpu.VMEM((ZPT,), jnp.float32),
            pltpu.VMEM((ZPT,), jnp.float32),
            pltpu.VMEM((ROWS, EB), jnp.int32),
            pltpu.VMEM_SHARED((ACC_ROWS,), jnp.float32),
            pltpu.SemaphoreType.DMA,
        ],
    )
    return k(d_tt, d_aa, d_at, d_ta)


def _scatter_body(n_chunks, t_tt, t_at, t_aa, t_ta,
                  s_tt, d_tt, s_at, d_at, s_aa, d_aa, s_ta, d_ta,
                  o_tt, o_at, o_aa, o_ta,
                  sbuf, dbuf, gbuf, zbuf, fbuf, acc, gsem, ssem, fs0, fs1):
    cps = n_chunks // NC
    core = lax.axis_index("c")
    s = lax.axis_index("s")
    tabs = (t_tt, t_at, t_aa, t_ta)
    srcs = (s_tt, s_at, s_aa, s_ta)
    dsts = (d_tt, d_at, d_aa, d_ta)
    outs = (o_tt, o_at, o_aa, o_ta)

    def fill(i, _):
        zbuf[i % ZROWS, pl.ds((i // ZROWS) * 16, 16)] = jnp.zeros(
            (16,), jnp.float32)
        return _

    lax.fori_loop(0, ZROWS * (CW // 16), fill, None)

    for rel in range(4):
        pltpu.sync_copy(srcs[rel].at[s], sbuf)
        pltpu.sync_copy(dsts[rel].at[s], dbuf)

        def chunk(ci, _, rel=rel):
            def zero(zg, _):
                zd = [pltpu.async_copy(
                          zbuf, acc.at[pl.ds(s * ZPT + (zg * NB + b) * ZROWS,
                                             ZROWS)], gsem)
                      for b in range(NB)]
                for d in zd:
                    d.wait()
                return _

            lax.fori_loop(0, ZPT // ZROWS // NB, zero, None)

            off = jnp.where(ci == 0, core * cps * N, N).astype(jnp.int32)

            def offset(j, _):
                for k in range(EB // 16):
                    sl = pl.ds(k * 16, 16)
                    sbuf[j, sl] = sbuf[j, sl] + off
                return _

            lax.fori_loop(0, ROWS, offset, None)
            plsc.subcore_barrier()

            def edge(g, _, rel=rel):
                gd = [pltpu.async_copy(tabs[rel].at[sbuf.at[g * NB + b]],
                                       gbuf.at[b], gsem)
                      for b in range(NB)]
                for d in gd:
                    d.wait()
                sd = [pltpu.async_copy(gbuf.at[b],
                                       acc.at[dbuf.at[g * NB + b]], ssem,
                                       add=True)
                      for b in range(NB)]
                for d in sd:
                    d.wait()
                return _

            lax.fori_loop(0, ROWS // NB, edge, None)
            plsc.subcore_barrier()

            obase = (core * cps + ci) * ACC_ROWS + s * ZPT

            def flush(f, _, rel=rel):
                r0 = 2 * f * ZROWS
                pltpu.sync_copy(acc.at[pl.ds(s * ZPT + r0, ZROWS)],
                                fbuf.at[0])
                d0 = pltpu.async_copy(
                    fbuf.at[0], outs[rel].at[pl.ds(obase + r0, ZROWS)], fs0)
                pltpu.sync_copy(acc.at[pl.ds(s * ZPT + r0 + ZROWS, ZROWS)],
                                fbuf.at[1])
                d1 = pltpu.async_copy(
                    fbuf.at[1], outs[rel].at[pl.ds(obase + r0 + ZROWS,
                                                   ZROWS)], fs1)
                d0.wait()
                d1.wait()
                return _

            lax.fori_loop(0, ZPT // ZROWS // 2, flush, None)
            plsc.subcore_barrier()
            return _

        lax.fori_loop(0, cps, chunk, None)


def _sc_scatter(n_chunks, tabs, edges):
    out_t = tuple(jax.ShapeDtypeStruct((n_chunks * ACC_ROWS, CW), jnp.float32)
                  for _ in range(4))
    k = pl.kernel(
        functools.partial(_scatter_body, n_chunks),
        out_type=out_t,
        mesh=plsc.VectorSubcoreMesh(**_MESH),
        scratch_types=[
            pltpu.VMEM((ROWS, EB), jnp.int32),
            pltpu.VMEM((ROWS, EB), jnp.int32),
            pltpu.VMEM((NB, EB, CW), jnp.float32),
            pltpu.VMEM((ZROWS, CW), jnp.float32),
            pltpu.VMEM((2, ZROWS, CW), jnp.float32),
            pltpu.VMEM_SHARED((ACC_ROWS, CW), jnp.float32),
            pltpu.SemaphoreType.DMA,
            pltpu.SemaphoreType.DMA,
            pltpu.SemaphoreType.DMA,
            pltpu.SemaphoreType.DMA,
        ],
        compiler_params=pltpu.CompilerParams(use_tc_tiling_on_sc=False),
    )
    return k(*tabs, *edges)


def _mm1_body(x_ref, wg_ref, wl_ref, wr_ref, hist_ref, g_ref, p_ref, r_ref):
    x = x_ref[...]
    dinv = lax.rsqrt(hist_ref[...] + 1.0)
    g = jnp.dot(x, wg_ref[...], preferred_element_type=jnp.float32) * dinv
    p = jnp.dot(x, wl_ref[...], preferred_element_type=jnp.float32)
    r = jnp.dot(x, wr_ref[...], preferred_element_type=jnp.float32)
    for c in range(C1):
        sl = slice(c * CW, (c + 1) * CW)
        g_ref[c, :, :] = g[:, sl]
        p_ref[c, :, :] = p[:, sl]
        r_ref[c, :, :] = r[:, sl]


def _mm1(x, wg, wl, wr, hist):
    chunked = pl.BlockSpec((C1, R, CW), lambda i: (0, i, 0))
    return pl.pallas_call(
        _mm1_body,
        grid=(GRID,),
        in_specs=[
            pl.BlockSpec((R, D_IN), lambda i: (i, 0)),
            pl.BlockSpec((D_IN, D_H), lambda i: (0, 0)),
            pl.BlockSpec((D_IN, D_H), lambda i: (0, 0)),
            pl.BlockSpec((D_IN, D_H), lambda i: (0, 0)),
            pl.BlockSpec((R, 1), lambda i: (i, 0)),
        ],
        out_specs=[chunked, chunked, chunked],
        out_shape=[jax.ShapeDtypeStruct((C1, N, CW), jnp.float32)] * 3,
    )(x, wg, wl, wr, hist)


def _mm2_body(s_ref, g_ref, a_ref, rr_ref, hist_ref, cnt_ref, b_ref,
              wg_ref, wl_ref, wr_ref, g2_ref, p2_ref, r2_ref):
    dinv = lax.rsqrt(hist_ref[...] + 1.0)
    cinv = 1.0 / jnp.maximum(cnt_ref[...], 1.0)
    h2 = jnp.zeros((R, D_OUT), jnp.float32)
    p2 = jnp.zeros((R, D_OUT), jnp.float32)
    r2 = jnp.zeros((R, D_OUT), jnp.float32)
    for c in range(C1):
        piece = jnp.maximum(
            dinv * (s_ref[c, :, :] + g_ref[c, :, :])
            + a_ref[c, :, :] * cinv + rr_ref[c, :, :] + b_ref[c], 0.0)
        sl = slice(c * CW, (c + 1) * CW)
        h2 += jnp.dot(piece, wg_ref[sl, :], preferred_element_type=jnp.float32)
        p2 += jnp.dot(piece, wl_ref[sl, :], preferred_element_type=jnp.float32)
        r2 += jnp.dot(piece, wr_ref[sl, :], preferred_element_type=jnp.float32)
    g2 = dinv * h2
    for c in range(C2):
        sl = slice(c * CW, (c + 1) * CW)
        g2_ref[c, :, :] = g2[:, sl]
        p2_ref[c, :, :] = p2[:, sl]
    r2_ref[...] = r2


def _mm2(s, g, a, rr, hist, cnt, b, wg, wl, wr):
    in4 = pl.BlockSpec((C1, R, CW), lambda i: (0, i, 0))
    col = pl.BlockSpec((R, 1), lambda i: (i, 0))
    w = pl.BlockSpec((D_H, D_OUT), lambda i: (0, 0))
    out2 = pl.BlockSpec((C2, R, CW), lambda i: (0, i, 0))
    return pl.pallas_call(
        _mm2_body,
        grid=(GRID,),
        in_specs=[in4, in4, in4, in4, col, col,
                  pl.BlockSpec((C1, CW), lambda i: (0, 0)), w, w, w],
        out_specs=[out2, out2, pl.BlockSpec((R, D_OUT), lambda i: (i, 0))],
        out_shape=[
            jax.ShapeDtypeStruct((C2, N, CW), jnp.float32),
            jax.ShapeDtypeStruct((C2, N, CW), jnp.float32),
            jax.ShapeDtypeStruct((N, D_OUT), jnp.float32),
        ],
    )(s, g, a, rr, hist, cnt, b, wg, wl, wr)


def _fin_body(s_ref, g_ref, a_ref, r2_ref, hist_ref, cnt_ref, b_ref, out_ref):
    dinv = lax.rsqrt(hist_ref[...] + 1.0)
    cinv = 1.0 / jnp.maximum(cnt_ref[...], 1.0)
    for c in range(C2):
        sl = slice(c * CW, (c + 1) * CW)
        out_ref[:, sl] = (dinv * (s_ref[c, :, :] + g_ref[c, :, :])
                          + a_ref[c, :, :] * cinv + b_ref[c]
                          + r2_ref[:, sl])


def _fin(s, g, a, r2, hist, cnt, b):
    in2 = pl.BlockSpec((C2, R, CW), lambda i: (0, i, 0))
    col = pl.BlockSpec((R, 1), lambda i: (i, 0))
    return pl.pallas_call(
        _fin_body,
        grid=(GRID,),
        in_specs=[in2, in2, in2,
                  pl.BlockSpec((R, D_OUT), lambda i: (i, 0)), col, col,
                  pl.BlockSpec((C2, CW), lambda i: (0, 0))],
        out_specs=pl.BlockSpec((R, D_OUT), lambda i: (i, 0)),
        out_shape=jax.ShapeDtypeStruct((N, D_OUT), jnp.float32),
    )(s, g, a, r2, hist, cnt, b)


def _prep_edges(ei):
    pad = E_PAD - E
    src = jnp.concatenate([ei[0], jnp.zeros((pad,), jnp.int32)])
    dst = jnp.concatenate([ei[1], jnp.full((pad,), DUMP, jnp.int32)])
    return src.reshape(NS, ROWS, EB), dst.reshape(NS, ROWS, EB)


def kernel(x_tx, x_addr, ei_tx_tx, ei_addr_addr, ei_addr_tx, ei_tx_addr,
           l1_tt_W, l1_tt_b, l1_aa_W, l1_aa_b,
           l1_at_Wl, l1_at_bl, l1_at_Wr,
           l1_ta_Wl, l1_ta_bl, l1_ta_Wr,
           l2_tt_W, l2_tt_b, l2_aa_W, l2_aa_b,
           l2_at_Wl, l2_at_bl, l2_at_Wr,
           l2_ta_Wl, l2_ta_bl, l2_ta_Wr):
    s_tt, d_tt = _prep_edges(ei_tx_tx)
    s_aa, d_aa = _prep_edges(ei_addr_addr)
    s_at, d_at = _prep_edges(ei_addr_tx)
    s_ta, d_ta = _prep_edges(ei_tx_addr)
    edges = (s_tt, d_tt, s_at, d_at, s_aa, d_aa, s_ta, d_ta)

    hist = _hist_kernel(d_tt, d_aa, d_at, d_ta).reshape(4, ACC_ROWS)
    h_tt = hist[0, :N].reshape(N, 1)
    h_aa = hist[1, :N].reshape(N, 1)
    c_at = hist[2, :N].reshape(N, 1)
    c_ta = hist[3, :N].reshape(N, 1)

    g_t, p_t, r_t = _mm1(x_tx, l1_tt_W, l1_ta_Wl, l1_at_Wr, h_tt)
    g_a, p_a, r_a = _mm1(x_addr, l1_aa_W, l1_at_Wl, l1_ta_Wr, h_aa)

    tabs1 = (g_t.reshape(C1 * N, CW), p_a.reshape(C1 * N, CW),
             g_a.reshape(C1 * N, CW), p_t.reshape(C1 * N, CW))
    S_tt, A_at, S_aa, A_ta = _sc_scatter(C1, tabs1, edges)

    b1_t = (l1_tt_b + l1_at_bl).reshape(C1, CW)
    b1_a = (l1_aa_b + l1_ta_bl).reshape(C1, CW)
    g2_t, p2_t, r2_t = _mm2(S_tt.reshape(C1, ACC_ROWS, CW), g_t,
                            A_at.reshape(C1, ACC_ROWS, CW), r_t, h_tt, c_at,
                            b1_t, l2_tt_W, l2_ta_Wl, l2_at_Wr)
    g2_a, p2_a, r2_a = _mm2(S_aa.reshape(C1, ACC_ROWS, CW), g_a,
                            A_ta.reshape(C1, ACC_ROWS, CW), r_a, h_aa, c_ta,
                            b1_a, l2_aa_W, l2_at_Wl, l2_ta_Wr)

    tabs2 = (g2_t.reshape(C2 * N, CW), p2_a.reshape(C2 * N, CW),
             g2_a.reshape(C2 * N, CW), p2_t.reshape(C2 * N, CW))
    S2_tt, A2_at, S2_aa, A2_ta = _sc_scatter(C2, tabs2, edges)

    b2_t = (l2_tt_b + l2_at_bl).reshape(C2, CW)
    b2_a = (l2_aa_b + l2_ta_bl).reshape(C2, CW)
    t2 = _fin(S2_tt.reshape(C2, ACC_ROWS, CW), g2_t,
              A2_at.reshape(C2, ACC_ROWS, CW), r2_t, h_tt, c_at, b2_t)
    a2 = _fin(S2_aa.reshape(C2, ACC_ROWS, CW), g2_a,
              A2_ta.reshape(C2, ACC_ROWS, CW), r2_a, h_aa, c_ta, b2_a)
    return (t2, a2)


# speedup vs baseline: 10.7153x; 1.9099x over previous
import functools

import jax
import jax.numpy as jnp
from jax import lax
from jax.experimental import pallas as pl
from jax.experimental.pallas import tpu as pltpu
from jax.experimental.pallas import tpu_sc as plsc

N = 50000
E = 400000
D_IN = 128
D_H = 128
D_OUT = 64

NC = 2
NS = 16
CW = 16

EB = 128
ROWS = 196
EPT_PAD = ROWS * EB
E_PAD = NS * EPT_PAD
DUMP = N
ACC_ROWS = 50176
ZPT = ACC_ROWS // NS
ZROWS = 112
NB = 7

C1 = D_H // CW
C2 = D_OUT // CW
R = 1000
GRID = N // R

_MESH = dict(core_axis_name="c", subcore_axis_name="s")


def _hist_body(d_tt, d_aa, d_at, d_ta, hist_out, ones_v, zbuf, fbuf, dbuf,
               acc, hsem):
    core = lax.axis_index("c")
    s = lax.axis_index("s")
    d_refs = (d_tt, d_aa, d_at, d_ta)

    def fill(i, _):
        ones_v[pl.ds(i * 16, 16)] = jnp.ones((16,), jnp.float32)
        zbuf[pl.ds(i * 16, 16)] = jnp.zeros((16,), jnp.float32)
        return _

    lax.fori_loop(0, ZPT // 16, fill, None)

    def do_rel(rel):
        pltpu.sync_copy(zbuf, acc.at[pl.ds(s * ZPT, ZPT)])
        plsc.subcore_barrier()
        pltpu.sync_copy(d_refs[rel].at[s], dbuf)

        def step(g, _):
            ds = [pltpu.async_copy(ones_v.at[pl.ds(0, EB)],
                                   acc.at[dbuf.at[g * NB + b]], hsem,
                                   add=True)
                  for b in range(NB)]
            for d in ds:
                d.wait()
            return _

        lax.fori_loop(0, ROWS // NB, step, None)
        plsc.subcore_barrier()
        pltpu.sync_copy(acc.at[pl.ds(s * ZPT, ZPT)], fbuf)
        pltpu.sync_copy(fbuf, hist_out.at[pl.ds(rel * ACC_ROWS + s * ZPT,
                                                ZPT)])
        plsc.subcore_barrier()

    for c_py in range(NC):
        @pl.when(core == c_py)
        def _():
            for rel in (c_py, c_py + 2):
                do_rel(rel)


def _hist_kernel(d_tt, d_aa, d_at, d_ta):
    k = pl.kernel(
        _hist_body,
        out_type=jax.ShapeDtypeStruct((4 * ACC_ROWS,), jnp.float32),
        mesh=plsc.VectorSubcoreMesh(**_MESH),
        scratch_types=[
            pltpu.VMEM((ZPT,), jnp.float32),
            pltpu.VMEM((ZPT,), jnp.float32),
            pltpu.VMEM((ZPT,), jnp.float32),
            pltpu.VMEM((ROWS, EB), jnp.int32),
            pltpu.VMEM_SHARED((ACC_ROWS,), jnp.float32),
            pltpu.SemaphoreType.DMA,
        ],
    )
    return k(d_tt, d_aa, d_at, d_ta)


def _scatter_body(nch, t_tt, t_at, t_aa, t_ta,
                  s_tt, d_tt, s_at, d_at, s_aa, d_aa, s_ta, d_ta,
                  o_tt, o_at, o_aa, o_ta,
                  sbuf, dbuf, gbuf, zbuf, fbuf, ibuf, ionch, acc,
                  gsem, ssem, fs0, fs1):
    cps = nch // NC
    core = lax.axis_index("c")
    s = lax.axis_index("s")
    tabs = (t_tt, t_at, t_aa, t_ta)
    srcs = (s_tt, s_at, s_aa, s_ta)
    dsts = (d_tt, d_at, d_aa, d_ta)
    outs = (o_tt, o_at, o_aa, o_ta)

    def fill(i, _):
        zbuf[i % ZROWS, pl.ds((i // ZROWS) * 16, 16)] = jnp.zeros(
            (16,), jnp.float32)
        return _

    lax.fori_loop(0, ZROWS * (CW // 16), fill, None)

    def fill_iota(k, _):
        ionch[pl.ds(k * 16, 16)] = (jnp.arange(16, dtype=jnp.int32)
                                    + k * 16) * nch
        return _

    lax.fori_loop(0, ZROWS // 16, fill_iota, None)

    for rel in range(4):
        pltpu.sync_copy(srcs[rel].at[s], sbuf)
        pltpu.sync_copy(dsts[rel].at[s], dbuf)

        def chunk(ci, _, rel=rel):
            def zero(zg, _):
                zd = [pltpu.async_copy(
                          zbuf, acc.at[pl.ds(s * ZPT + (zg * NB + b) * ZROWS,
                                             ZROWS)], gsem)
                      for b in range(NB)]
                for d in zd:
                    d.wait()
                return _

            lax.fori_loop(0, ZPT // ZROWS // NB, zero, None)

            first = ci == 0

            def offset(j, _):
                for k in range(EB // 16):
                    sl = pl.ds(k * 16, 16)
                    v = sbuf[j, sl]
                    sbuf[j, sl] = jnp.where(first, v * nch + core * cps,
                                            v + 1)
                return _

            lax.fori_loop(0, ROWS, offset, None)
            plsc.subcore_barrier()

            def edge(g, _, rel=rel):
                gd = [pltpu.async_copy(tabs[rel].at[sbuf.at[g * NB + b]],
                                       gbuf.at[b], gsem)
                      for b in range(NB)]
                sd = []
                for b in range(NB):
                    gd[b].wait()
                    sd.append(pltpu.async_copy(
                        gbuf.at[b], acc.at[dbuf.at[g * NB + b]], ssem,
                        add=True))
                for d in sd:
                    d.wait()
                return _

            lax.fori_loop(0, ROWS // NB, edge, None)
            plsc.subcore_barrier()

            c_abs = core * cps + ci

            def flush(f, _, rel=rel):
                for b in range(2):
                    r0 = (2 * f + b) * ZROWS
                    base = (s * ZPT + r0) * nch + c_abs

                    def mkidx(k, _):
                        sl = pl.ds(k * 16, 16)
                        ibuf[b, sl] = ionch[sl] + base
                        return _

                    lax.fori_loop(0, ZROWS // 16, mkidx, None)
                    pltpu.sync_copy(acc.at[pl.ds(s * ZPT + r0, ZROWS)],
                                    fbuf.at[b])
                d0 = pltpu.async_copy(fbuf.at[0], outs[rel].at[ibuf.at[0]],
                                      fs0)
                d1 = pltpu.async_copy(fbuf.at[1], outs[rel].at[ibuf.at[1]],
                                      fs1)
                d0.wait()
                d1.wait()
                return _

            lax.fori_loop(0, ZPT // ZROWS // 2, flush, None)
            plsc.subcore_barrier()
            return _

        lax.fori_loop(0, cps, chunk, None)


def _sc_scatter(nch, tabs, edges):
    out_t = tuple(jax.ShapeDtypeStruct((nch * ACC_ROWS, CW), jnp.float32)
                  for _ in range(4))
    k = pl.kernel(
        functools.partial(_scatter_body, nch),
        out_type=out_t,
        mesh=plsc.VectorSubcoreMesh(**_MESH),
        scratch_types=[
            pltpu.VMEM((ROWS, EB), jnp.int32),
            pltpu.VMEM((ROWS, EB), jnp.int32),
            pltpu.VMEM((NB, EB, CW), jnp.float32),
            pltpu.VMEM((ZROWS, CW), jnp.float32),
            pltpu.VMEM((2, ZROWS, CW), jnp.float32),
            pltpu.VMEM((2, ZROWS), jnp.int32),
            pltpu.VMEM((ZROWS,), jnp.int32),
            pltpu.VMEM_SHARED((ACC_ROWS, CW), jnp.float32),
            pltpu.SemaphoreType.DMA,
            pltpu.SemaphoreType.DMA,
            pltpu.SemaphoreType.DMA,
            pltpu.SemaphoreType.DMA,
        ],
        compiler_params=pltpu.CompilerParams(use_tc_tiling_on_sc=False),
    )
    return k(*tabs, *edges)


def _mm1_body(x_ref, wg_ref, wl_ref, wr_ref, hist_ref, g_ref, p_ref, r_ref):
    x = x_ref[...]
    dinv = lax.rsqrt(hist_ref[...] + 1.0)
    g_ref[...] = jnp.dot(x, wg_ref[...],
                         preferred_element_type=jnp.float32) * dinv
    p_ref[...] = jnp.dot(x, wl_ref[...], preferred_element_type=jnp.float32)
    r_ref[...] = jnp.dot(x, wr_ref[...], preferred_element_type=jnp.float32)


def _mm1(x, wg, wl, wr, hist):
    blk = pl.BlockSpec((R, D_H), lambda i: (i, 0))
    return pl.pallas_call(
        _mm1_body,
        grid=(GRID,),
        in_specs=[
            pl.BlockSpec((R, D_IN), lambda i: (i, 0)),
            pl.BlockSpec((D_IN, D_H), lambda i: (0, 0)),
            pl.BlockSpec((D_IN, D_H), lambda i: (0, 0)),
            pl.BlockSpec((D_IN, D_H), lambda i: (0, 0)),
            pl.BlockSpec((R, 1), lambda i: (i, 0)),
        ],
        out_specs=[blk, blk, blk],
        out_shape=[jax.ShapeDtypeStruct((N, D_H), jnp.float32)] * 3,
    )(x, wg, wl, wr, hist)


def _mm2_body(s_ref, g_ref, a_ref, rr_ref, hist_ref, cnt_ref, b_ref,
              wg_ref, wl_ref, wr_ref, g2_ref, p2_ref, r2_ref):
    dinv = lax.rsqrt(hist_ref[...] + 1.0)
    cinv = 1.0 / jnp.maximum(cnt_ref[...], 1.0)
    t1 = jnp.maximum(dinv * (s_ref[...] + g_ref[...]) + a_ref[...] * cinv
                     + rr_ref[...] + b_ref[...], 0.0)
    h2 = jnp.dot(t1, wg_ref[...], preferred_element_type=jnp.float32)
    g2_ref[...] = h2 * dinv
    p2_ref[...] = jnp.dot(t1, wl_ref[...], preferred_element_type=jnp.float32)
    r2_ref[...] = jnp.dot(t1, wr_ref[...], preferred_element_type=jnp.float32)


def _mm2(s, g, a, rr, hist, cnt, b, wg, wl, wr):
    big = pl.BlockSpec((R, D_H), lambda i: (i, 0))
    col = pl.BlockSpec((R, 1), lambda i: (i, 0))
    w = pl.BlockSpec((D_H, D_OUT), lambda i: (0, 0))
    out = pl.BlockSpec((R, D_OUT), lambda i: (i, 0))
    return pl.pallas_call(
        _mm2_body,
        grid=(GRID,),
        in_specs=[big, big, big, big, col, col,
                  pl.BlockSpec((1, D_H), lambda i: (0, 0)), w, w, w],
        out_specs=[out, out, out],
        out_shape=[jax.ShapeDtypeStruct((N, D_OUT), jnp.float32)] * 3,
    )(s, g, a, rr, hist, cnt, b, wg, wl, wr)


def _fin_body(s_ref, g_ref, a_ref, r2_ref, hist_ref, cnt_ref, b_ref, out_ref):
    dinv = lax.rsqrt(hist_ref[...] + 1.0)
    cinv = 1.0 / jnp.maximum(cnt_ref[...], 1.0)
    out_ref[...] = (dinv * (s_ref[...] + g_ref[...]) + a_ref[...] * cinv
                    + b_ref[...] + r2_ref[...])


def _fin(s, g, a, r2, hist, cnt, b):
    sm = pl.BlockSpec((R, D_OUT), lambda i: (i, 0))
    col = pl.BlockSpec((R, 1), lambda i: (i, 0))
    return pl.pallas_call(
        _fin_body,
        grid=(GRID,),
        in_specs=[sm, sm, sm, sm, col, col,
                  pl.BlockSpec((1, D_OUT), lambda i: (0, 0))],
        out_specs=sm,
        out_shape=jax.ShapeDtypeStruct((N, D_OUT), jnp.float32),
    )(s, g, a, r2, hist, cnt, b)


def _prep_edges(ei):
    pad = E_PAD - E
    src = jnp.concatenate([ei[0], jnp.zeros((pad,), jnp.int32)])
    dst = jnp.concatenate([ei[1], jnp.full((pad,), DUMP, jnp.int32)])
    return src.reshape(NS, ROWS, EB), dst.reshape(NS, ROWS, EB)


def kernel(x_tx, x_addr, ei_tx_tx, ei_addr_addr, ei_addr_tx, ei_tx_addr,
           l1_tt_W, l1_tt_b, l1_aa_W, l1_aa_b,
           l1_at_Wl, l1_at_bl, l1_at_Wr,
           l1_ta_Wl, l1_ta_bl, l1_ta_Wr,
           l2_tt_W, l2_tt_b, l2_aa_W, l2_aa_b,
           l2_at_Wl, l2_at_bl, l2_at_Wr,
           l2_ta_Wl, l2_ta_bl, l2_ta_Wr):
    s_tt, d_tt = _prep_edges(ei_tx_tx)
    s_aa, d_aa = _prep_edges(ei_addr_addr)
    s_at, d_at = _prep_edges(ei_addr_tx)
    s_ta, d_ta = _prep_edges(ei_tx_addr)
    edges = (s_tt, d_tt, s_at, d_at, s_aa, d_aa, s_ta, d_ta)

    hist = _hist_kernel(d_tt, d_aa, d_at, d_ta).reshape(4, ACC_ROWS)
    h_tt = hist[0, :N].reshape(N, 1)
    h_aa = hist[1, :N].reshape(N, 1)
    c_at = hist[2, :N].reshape(N, 1)
    c_ta = hist[3, :N].reshape(N, 1)

    g_t, p_t, r_t = _mm1(x_tx, l1_tt_W, l1_ta_Wl, l1_at_Wr, h_tt)
    g_a, p_a, r_a = _mm1(x_addr, l1_aa_W, l1_at_Wl, l1_ta_Wr, h_aa)

    tabs1 = (g_t.reshape(C1 * N, CW), p_a.reshape(C1 * N, CW),
             g_a.reshape(C1 * N, CW), p_t.reshape(C1 * N, CW))
    S_tt, A_at, S_aa, A_ta = _sc_scatter(C1, tabs1, edges)

    b1_t = (l1_tt_b + l1_at_bl).reshape(1, D_H)
    b1_a = (l1_aa_b + l1_ta_bl).reshape(1, D_H)
    g2_t, p2_t, r2_t = _mm2(S_tt.reshape(ACC_ROWS, D_H), g_t,
                            A_at.reshape(ACC_ROWS, D_H), r_t, h_tt, c_at,
                            b1_t, l2_tt_W, l2_ta_Wl, l2_at_Wr)
    g2_a, p2_a, r2_a = _mm2(S_aa.reshape(ACC_ROWS, D_H), g_a,
                            A_ta.reshape(ACC_ROWS, D_H), r_a, h_aa, c_ta,
                            b1_a, l2_aa_W, l2_at_Wl, l2_ta_Wr)

    tabs2 = (g2_t.reshape(C2 * N, CW), p2_a.reshape(C2 * N, CW),
             g2_a.reshape(C2 * N, CW), p2_t.reshape(C2 * N, CW))
    S2_tt, A2_at, S2_aa, A2_ta = _sc_scatter(C2, tabs2, edges)

    b2_t = (l2_tt_b + l2_at_bl).reshape(1, D_OUT)
    b2_a = (l2_aa_b + l2_ta_bl).reshape(1, D_OUT)
    t2 = _fin(S2_tt.reshape(ACC_ROWS, D_OUT), g2_t,
              A2_at.reshape(ACC_ROWS, D_OUT), r2_t, h_tt, c_at, b2_t)
    a2 = _fin(S2_aa.reshape(ACC_ROWS, D_OUT), g2_a,
              A2_ta.reshape(ACC_ROWS, D_OUT), r2_a, h_aa, c_ta, b2_a)
    return (t2, a2)


# speedup vs baseline: 11.3244x; 1.0568x over previous
import functools

import jax
import jax.numpy as jnp
from jax import lax
from jax.experimental import pallas as pl
from jax.experimental.pallas import tpu as pltpu
from jax.experimental.pallas import tpu_sc as plsc

N = 50000
E = 400000
D_IN = 128
D_H = 128
D_OUT = 64

NC = 2
NS = 16
CW = 16

EB = 128
ROWS = 196
EPT_PAD = ROWS * EB
E_PAD = NS * EPT_PAD
DUMP = N
ACC_ROWS = 50176
ZPT = ACC_ROWS // NS
ZROWS = 112
NB = 7
NBP = 4
GROUPS = ROWS // NBP
BANKS = 3

C1 = D_H // CW
C2 = D_OUT // CW
R = 1000
GRID = N // R

_MESH = dict(core_axis_name="c", subcore_axis_name="s")


def _hist_body(d_tt, d_aa, d_at, d_ta, hist_out, ones_v, zbuf, fbuf, dbuf,
               acc, hsem):
    core = lax.axis_index("c")
    s = lax.axis_index("s")
    d_refs = (d_tt, d_aa, d_at, d_ta)

    def fill(i, _):
        ones_v[pl.ds(i * 16, 16)] = jnp.ones((16,), jnp.float32)
        zbuf[pl.ds(i * 16, 16)] = jnp.zeros((16,), jnp.float32)
        return _

    lax.fori_loop(0, ZPT // 16, fill, None)

    def do_rel(rel):
        pltpu.sync_copy(zbuf, acc.at[pl.ds(s * ZPT, ZPT)])
        plsc.subcore_barrier()
        pltpu.sync_copy(d_refs[rel].at[s], dbuf)

        def step(g, _):
            ds = [pltpu.async_copy(ones_v.at[pl.ds(0, EB)],
                                   acc.at[dbuf.at[g * NB + b]], hsem,
                                   add=True)
                  for b in range(NB)]
            for d in ds:
                d.wait()
            return _

        lax.fori_loop(0, ROWS // NB, step, None)
        plsc.subcore_barrier()
        pltpu.sync_copy(acc.at[pl.ds(s * ZPT, ZPT)], fbuf)
        pltpu.sync_copy(fbuf, hist_out.at[pl.ds(rel * ACC_ROWS + s * ZPT,
                                                ZPT)])
        plsc.subcore_barrier()

    for c_py in range(NC):
        @pl.when(core == c_py)
        def _():
            for rel in (c_py, c_py + 2):
                do_rel(rel)


def _hist_kernel(d_tt, d_aa, d_at, d_ta):
    k = pl.kernel(
        _hist_body,
        out_type=jax.ShapeDtypeStruct((4 * ACC_ROWS,), jnp.float32),
        mesh=plsc.VectorSubcoreMesh(**_MESH),
        scratch_types=[
            pltpu.VMEM((ZPT,), jnp.float32),
            pltpu.VMEM((ZPT,), jnp.float32),
            pltpu.VMEM((ZPT,), jnp.float32),
            pltpu.VMEM((ROWS, EB), jnp.int32),
            pltpu.VMEM_SHARED((ACC_ROWS,), jnp.float32),
            pltpu.SemaphoreType.DMA,
        ],
    )
    return k(d_tt, d_aa, d_at, d_ta)


def _scatter_body(nch, t_tt, t_at, t_aa, t_ta,
                  s_tt, d_tt, s_at, d_at, s_aa, d_aa, s_ta, d_ta,
                  o_tt, o_at, o_aa, o_ta,
                  sbuf, dbuf, gbuf, fbuf, ibuf, ionch, acc,
                  gsem, ssem, fs0, fs1):
    cps = nch // NC
    core = lax.axis_index("c")
    s = lax.axis_index("s")
    tabs = (t_tt, t_at, t_aa, t_ta)
    srcs = (s_tt, s_at, s_aa, s_ta)
    dsts = (d_tt, d_at, d_aa, d_ta)
    outs = (o_tt, o_at, o_aa, o_ta)

    def fill_iota(k, _):
        ionch[pl.ds(k * 16, 16)] = (jnp.arange(16, dtype=jnp.int32)
                                    + k * 16) * nch
        return _

    lax.fori_loop(0, ZROWS // 16, fill_iota, None)

    for rel in range(4):
        pltpu.sync_copy(srcs[rel].at[s], sbuf)
        pltpu.sync_copy(dsts[rel].at[s], dbuf)

        def chunk(ci, _, rel=rel):
            def zf(i, _):
                fbuf[0, i % ZROWS, pl.ds((i // ZROWS) * 16, 16)] = (
                    jnp.zeros((16,), jnp.float32))
                return _

            lax.fori_loop(0, ZROWS * (CW // 16), zf, None)
            zd = [pltpu.async_copy(
                      fbuf.at[0], acc.at[pl.ds(s * ZPT + z * ZROWS, ZROWS)],
                      gsem)
                  for z in range(ZPT // ZROWS)]

            first = ci == 0

            def offset(j, _):
                for k in range(EB // 16):
                    sl = pl.ds(k * 16, 16)
                    v = sbuf[j, sl]
                    sbuf[j, sl] = jnp.where(first, v * nch + core * cps,
                                            v + 1)
                return _

            lax.fori_loop(0, ROWS, offset, None)
            for d in zd:
                d.wait()
            plsc.subcore_barrier()

            def gfire(g, bk):
                return [pltpu.async_copy(
                            tabs[rel].at[sbuf.at[g * NBP + b]],
                            gbuf.at[bk, b], gsem)
                        for b in range(NBP)]

            def gwait(g, bk):
                for b in range(NBP):
                    pltpu.make_async_copy(
                        tabs[rel].at[sbuf.at[g * NBP + b]],
                        gbuf.at[bk, b], gsem).wait()

            def sfire(g, bk):
                return [pltpu.async_copy(
                            gbuf.at[bk, b], acc.at[dbuf.at[g * NBP + b]],
                            ssem, add=True)
                        for b in range(NBP)]

            def swait(g, bk):
                for b in range(NBP):
                    pltpu.make_async_copy(
                        gbuf.at[bk, b], acc.at[dbuf.at[g * NBP + b]],
                        ssem).wait()

            gfire(0, 0)
            gfire(1, 1)

            def edge(g, _, rel=rel):
                bk = g % 3
                gwait(g, bk)
                sfire(g, bk)
                gp = jnp.maximum(g - 1, 0)

                @pl.when(g >= 1)
                def _():
                    swait(gp, gp % 3)

                gn = jnp.minimum(g + 2, GROUPS - 1)

                @pl.when(g + 2 <= GROUPS - 1)
                def _():
                    gfire(gn, gn % 3)

                return _

            lax.fori_loop(0, GROUPS, edge, None)
            swait(GROUPS - 1, (GROUPS - 1) % 3)
            plsc.subcore_barrier()

            c_abs = core * cps + ci

            def flush(f, _, rel=rel):
                for b in range(2):
                    r0 = (2 * f + b) * ZROWS
                    base = (s * ZPT + r0) * nch + c_abs

                    def mkidx(k, _):
                        sl = pl.ds(k * 16, 16)
                        ibuf[b, sl] = ionch[sl] + base
                        return _

                    lax.fori_loop(0, ZROWS // 16, mkidx, None)
                    pltpu.sync_copy(acc.at[pl.ds(s * ZPT + r0, ZROWS)],
                                    fbuf.at[b])
                d0 = pltpu.async_copy(fbuf.at[0], outs[rel].at[ibuf.at[0]],
                                      fs0)
                d1 = pltpu.async_copy(fbuf.at[1], outs[rel].at[ibuf.at[1]],
                                      fs1)
                d0.wait()
                d1.wait()
                return _

            lax.fori_loop(0, ZPT // ZROWS // 2, flush, None)
            plsc.subcore_barrier()
            return _

        lax.fori_loop(0, cps, chunk, None)


def _sc_scatter(nch, tabs, edges):
    out_t = tuple(jax.ShapeDtypeStruct((nch * ACC_ROWS, CW), jnp.float32)
                  for _ in range(4))
    k = pl.kernel(
        functools.partial(_scatter_body, nch),
        out_type=out_t,
        mesh=plsc.VectorSubcoreMesh(**_MESH),
        scratch_types=[
            pltpu.VMEM((ROWS, EB), jnp.int32),
            pltpu.VMEM((ROWS, EB), jnp.int32),
            pltpu.VMEM((BANKS, NBP, EB, CW), jnp.float32),
            pltpu.VMEM((2, ZROWS, CW), jnp.float32),
            pltpu.VMEM((2, ZROWS), jnp.int32),
            pltpu.VMEM((ZROWS,), jnp.int32),
            pltpu.VMEM_SHARED((ACC_ROWS, CW), jnp.float32),
            pltpu.SemaphoreType.DMA,
            pltpu.SemaphoreType.DMA,
            pltpu.SemaphoreType.DMA,
            pltpu.SemaphoreType.DMA,
        ],
        compiler_params=pltpu.CompilerParams(use_tc_tiling_on_sc=False),
    )
    return k(*tabs, *edges)


def _mm1_body(x_ref, wg_ref, wl_ref, wr_ref, hist_ref, g_ref, p_ref, r_ref):
    x = x_ref[...]
    dinv = lax.rsqrt(hist_ref[...] + 1.0)
    g_ref[...] = jnp.dot(x, wg_ref[...],
                         preferred_element_type=jnp.float32) * dinv
    p_ref[...] = jnp.dot(x, wl_ref[...], preferred_element_type=jnp.float32)
    r_ref[...] = jnp.dot(x, wr_ref[...], preferred_element_type=jnp.float32)


def _mm1(x, wg, wl, wr, hist):
    blk = pl.BlockSpec((R, D_H), lambda i: (i, 0))
    return pl.pallas_call(
        _mm1_body,
        grid=(GRID,),
        in_specs=[
            pl.BlockSpec((R, D_IN), lambda i: (i, 0)),
            pl.BlockSpec((D_IN, D_H), lambda i: (0, 0)),
            pl.BlockSpec((D_IN, D_H), lambda i: (0, 0)),
            pl.BlockSpec((D_IN, D_H), lambda i: (0, 0)),
            pl.BlockSpec((R, 1), lambda i: (i, 0)),
        ],
        out_specs=[blk, blk, blk],
        out_shape=[jax.ShapeDtypeStruct((N, D_H), jnp.float32)] * 3,
    )(x, wg, wl, wr, hist)


def _mm2_body(s_ref, g_ref, a_ref, rr_ref, hist_ref, cnt_ref, b_ref,
              wg_ref, wl_ref, wr_ref, g2_ref, p2_ref, r2_ref):
    dinv = lax.rsqrt(hist_ref[...] + 1.0)
    cinv = 1.0 / jnp.maximum(cnt_ref[...], 1.0)
    t1 = jnp.maximum(dinv * (s_ref[...] + g_ref[...]) + a_ref[...] * cinv
                     + rr_ref[...] + b_ref[...], 0.0)
    h2 = jnp.dot(t1, wg_ref[...], preferred_element_type=jnp.float32)
    g2_ref[...] = h2 * dinv
    p2_ref[...] = jnp.dot(t1, wl_ref[...], preferred_element_type=jnp.float32)
    r2_ref[...] = jnp.dot(t1, wr_ref[...], preferred_element_type=jnp.float32)


def _mm2(s, g, a, rr, hist, cnt, b, wg, wl, wr):
    big = pl.BlockSpec((R, D_H), lambda i: (i, 0))
    col = pl.BlockSpec((R, 1), lambda i: (i, 0))
    w = pl.BlockSpec((D_H, D_OUT), lambda i: (0, 0))
    out = pl.BlockSpec((R, D_OUT), lambda i: (i, 0))
    return pl.pallas_call(
        _mm2_body,
        grid=(GRID,),
        in_specs=[big, big, big, big, col, col,
                  pl.BlockSpec((1, D_H), lambda i: (0, 0)), w, w, w],
        out_specs=[out, out, out],
        out_shape=[jax.ShapeDtypeStruct((N, D_OUT), jnp.float32)] * 3,
    )(s, g, a, rr, hist, cnt, b, wg, wl, wr)


def _fin_body(s_ref, g_ref, a_ref, r2_ref, hist_ref, cnt_ref, b_ref, out_ref):
    dinv = lax.rsqrt(hist_ref[...] + 1.0)
    cinv = 1.0 / jnp.maximum(cnt_ref[...], 1.0)
    out_ref[...] = (dinv * (s_ref[...] + g_ref[...]) + a_ref[...] * cinv
                    + b_ref[...] + r2_ref[...])


def _fin(s, g, a, r2, hist, cnt, b):
    sm = pl.BlockSpec((R, D_OUT), lambda i: (i, 0))
    col = pl.BlockSpec((R, 1), lambda i: (i, 0))
    return pl.pallas_call(
        _fin_body,
        grid=(GRID,),
        in_specs=[sm, sm, sm, sm, col, col,
                  pl.BlockSpec((1, D_OUT), lambda i: (0, 0))],
        out_specs=sm,
        out_shape=jax.ShapeDtypeStruct((N, D_OUT), jnp.float32),
    )(s, g, a, r2, hist, cnt, b)


def _prep_edges(ei):
    pad = E_PAD - E
    src = jnp.concatenate([ei[0], jnp.zeros((pad,), jnp.int32)])
    dst = jnp.concatenate([ei[1], jnp.full((pad,), DUMP, jnp.int32)])
    return src.reshape(NS, ROWS, EB), dst.reshape(NS, ROWS, EB)


def kernel(x_tx, x_addr, ei_tx_tx, ei_addr_addr, ei_addr_tx, ei_tx_addr,
           l1_tt_W, l1_tt_b, l1_aa_W, l1_aa_b,
           l1_at_Wl, l1_at_bl, l1_at_Wr,
           l1_ta_Wl, l1_ta_bl, l1_ta_Wr,
           l2_tt_W, l2_tt_b, l2_aa_W, l2_aa_b,
           l2_at_Wl, l2_at_bl, l2_at_Wr,
           l2_ta_Wl, l2_ta_bl, l2_ta_Wr):
    s_tt, d_tt = _prep_edges(ei_tx_tx)
    s_aa, d_aa = _prep_edges(ei_addr_addr)
    s_at, d_at = _prep_edges(ei_addr_tx)
    s_ta, d_ta = _prep_edges(ei_tx_addr)
    edges = (s_tt, d_tt, s_at, d_at, s_aa, d_aa, s_ta, d_ta)

    hist = _hist_kernel(d_tt, d_aa, d_at, d_ta).reshape(4, ACC_ROWS)
    h_tt = hist[0, :N].reshape(N, 1)
    h_aa = hist[1, :N].reshape(N, 1)
    c_at = hist[2, :N].reshape(N, 1)
    c_ta = hist[3, :N].reshape(N, 1)

    g_t, p_t, r_t = _mm1(x_tx, l1_tt_W, l1_ta_Wl, l1_at_Wr, h_tt)
    g_a, p_a, r_a = _mm1(x_addr, l1_aa_W, l1_at_Wl, l1_ta_Wr, h_aa)

    tabs1 = (g_t.reshape(C1 * N, CW), p_a.reshape(C1 * N, CW),
             g_a.reshape(C1 * N, CW), p_t.reshape(C1 * N, CW))
    S_tt, A_at, S_aa, A_ta = _sc_scatter(C1, tabs1, edges)

    b1_t = (l1_tt_b + l1_at_bl).reshape(1, D_H)
    b1_a = (l1_aa_b + l1_ta_bl).reshape(1, D_H)
    g2_t, p2_t, r2_t = _mm2(S_tt.reshape(ACC_ROWS, D_H), g_t,
                            A_at.reshape(ACC_ROWS, D_H), r_t, h_tt, c_at,
                            b1_t, l2_tt_W, l2_ta_Wl, l2_at_Wr)
    g2_a, p2_a, r2_a = _mm2(S_aa.reshape(ACC_ROWS, D_H), g_a,
                            A_ta.reshape(ACC_ROWS, D_H), r_a, h_aa, c_ta,
                            b1_a, l2_aa_W, l2_at_Wl, l2_ta_Wr)

    tabs2 = (g2_t.reshape(C2 * N, CW), p2_a.reshape(C2 * N, CW),
             g2_a.reshape(C2 * N, CW), p2_t.reshape(C2 * N, CW))
    S2_tt, A2_at, S2_aa, A2_ta = _sc_scatter(C2, tabs2, edges)

    b2_t = (l2_tt_b + l2_at_bl).reshape(1, D_OUT)
    b2_a = (l2_aa_b + l2_ta_bl).reshape(1, D_OUT)
    t2 = _fin(S2_tt.reshape(ACC_ROWS, D_OUT), g2_t,
              A2_at.reshape(ACC_ROWS, D_OUT), r2_t, h_tt, c_at, b2_t)
    a2 = _fin(S2_aa.reshape(ACC_ROWS, D_OUT), g2_a,
              A2_ta.reshape(ACC_ROWS, D_OUT), r2_a, h_aa, c_ta, b2_a)
    return (t2, a2)


# speedup vs baseline: 11.5957x; 1.0240x over previous
import functools

import jax
import jax.numpy as jnp
from jax import lax
from jax.experimental import pallas as pl
from jax.experimental.pallas import tpu as pltpu
from jax.experimental.pallas import tpu_sc as plsc

N = 50000
E = 400000
D_IN = 128
D_H = 128
D_OUT = 64

NC = 2
NS = 16
CW = 32

EB = 128
ROWS = 196
EPT_PAD = ROWS * EB
E_PAD = NS * EPT_PAD
DUMP = N
ACC_ROWS = 50176
ZPT = ACC_ROWS // NS
ZROWS = 28
FPAD = 32
NZ = 7
NB = 7
SL = 28

C1 = D_H // CW
C2 = D_OUT // CW
R = 1000
GRID = N // R

_MESH = dict(core_axis_name="c", subcore_axis_name="s")


def _hist_body(d_tt, d_aa, d_at, d_ta, hist_out, ones_v, zbuf, fbuf, dbuf,
               acc, hsem):
    core = lax.axis_index("c")
    s = lax.axis_index("s")
    d_refs = (d_tt, d_aa, d_at, d_ta)

    def fill(i, _):
        ones_v[pl.ds(i * 16, 16)] = jnp.ones((16,), jnp.float32)
        zbuf[pl.ds(i * 16, 16)] = jnp.zeros((16,), jnp.float32)
        return _

    lax.fori_loop(0, ZPT // 16, fill, None)

    def do_rel(rel):
        pltpu.sync_copy(zbuf, acc.at[pl.ds(s * ZPT, ZPT)])
        plsc.subcore_barrier()
        pltpu.sync_copy(d_refs[rel].at[s], dbuf)

        def step(g, _):
            ds = [pltpu.async_copy(ones_v.at[pl.ds(0, EB)],
                                   acc.at[dbuf.at[g * NB + b]], hsem,
                                   add=True)
                  for b in range(NB)]
            for d in ds:
                d.wait()
            return _

        lax.fori_loop(0, ROWS // NB, step, None)
        plsc.subcore_barrier()
        pltpu.sync_copy(acc.at[pl.ds(s * ZPT, ZPT)], fbuf)
        pltpu.sync_copy(fbuf, hist_out.at[pl.ds(rel * ACC_ROWS + s * ZPT,
                                                ZPT)])
        plsc.subcore_barrier()

    for c_py in range(NC):
        @pl.when(core == c_py)
        def _():
            for rel in (c_py, c_py + 2):
                do_rel(rel)


def _hist_kernel(d_tt, d_aa, d_at, d_ta):
    k = pl.kernel(
        _hist_body,
        out_type=jax.ShapeDtypeStruct((4 * ACC_ROWS,), jnp.float32),
        mesh=plsc.VectorSubcoreMesh(**_MESH),
        scratch_types=[
            pltpu.VMEM((ZPT,), jnp.float32),
            pltpu.VMEM((ZPT,), jnp.float32),
            pltpu.VMEM((ZPT,), jnp.float32),
            pltpu.VMEM((ROWS, EB), jnp.int32),
            pltpu.VMEM_SHARED((ACC_ROWS,), jnp.float32),
            pltpu.SemaphoreType.DMA,
        ],
    )
    return k(d_tt, d_aa, d_at, d_ta)


def _scatter_body(nch, t_tt, t_at, t_aa, t_ta,
                  s_tt, d_tt, s_at, d_at, s_aa, d_aa, s_ta, d_ta,
                  o_tt, o_at, o_aa, o_ta,
                  sidx, didx, gbuf, fbuf, ibuf, ionch, acc,
                  gsem, ssem, isem, fs0, fs1):
    cps = nch // NC
    core = lax.axis_index("c")
    s = lax.axis_index("s")
    tabs = (t_tt, t_at, t_aa, t_ta)
    srcs = (s_tt, s_at, s_aa, s_ta)
    dsts = (d_tt, d_at, d_aa, d_ta)
    outs = (o_tt, o_at, o_aa, o_ta)

    def fill_iota(k, _):
        ionch[pl.ds(k * 16, 16)] = (jnp.arange(16, dtype=jnp.int32)
                                    + k * 16) * nch
        return _

    lax.fori_loop(0, FPAD // 16, fill_iota, None)

    for rel in range(4):
        def chunk(ci, _, rel=rel):
            c_abs = core * cps + ci

            def slab_fire(k):
                bk = k % 2
                return (pltpu.async_copy(
                            srcs[rel].at[s, pl.ds(k * SL, SL)],
                            sidx.at[bk], isem),
                        pltpu.async_copy(
                            dsts[rel].at[s, pl.ds(k * SL, SL)],
                            didx.at[bk], isem))

            def slab_wait(k):
                bk = k % 2
                pltpu.make_async_copy(srcs[rel].at[s, pl.ds(k * SL, SL)],
                                      sidx.at[bk], isem).wait()
                pltpu.make_async_copy(dsts[rel].at[s, pl.ds(k * SL, SL)],
                                      didx.at[bk], isem).wait()

            def offset_row(g):
                bk = (g // SL) % 2
                row = g % SL
                for k in range(EB // 16):
                    sl = pl.ds(k * 16, 16)
                    sidx[bk, row, sl] = sidx[bk, row, sl] * nch + c_abs

            def zf(i, _):
                fbuf[0, i % ZROWS, pl.ds((i // ZROWS) * 16, 16)] = (
                    jnp.zeros((16,), jnp.float32))
                return _

            lax.fori_loop(0, ZROWS * (CW // 16), zf, None)
            slab_fire(0)
            slab_fire(1)

            def zero(zg, _):
                zd = [pltpu.async_copy(
                          fbuf.at[0, pl.ds(0, ZROWS)],
                          acc.at[pl.ds(s * ZPT + (zg * NZ + z) * ZROWS,
                                       ZROWS)], gsem)
                      for z in range(NZ)]
                for d in zd:
                    d.wait()
                return _

            lax.fori_loop(0, ZPT // ZROWS // NZ, zero, None)
            plsc.subcore_barrier()

            def gfire(g):
                bk = (g // SL) % 2
                return pltpu.async_copy(
                    tabs[rel].at[sidx.at[bk, g % SL]], gbuf.at[g % 3], gsem)

            def gwait(g):
                bk = (g // SL) % 2
                pltpu.make_async_copy(
                    tabs[rel].at[sidx.at[bk, g % SL]], gbuf.at[g % 3],
                    gsem).wait()

            def sfire(g):
                bk = (g // SL) % 2
                return pltpu.async_copy(
                    gbuf.at[g % 3], acc.at[didx.at[bk, g % SL]], ssem,
                    add=True)

            def swait(g):
                bk = (g // SL) % 2
                pltpu.make_async_copy(
                    gbuf.at[g % 3], acc.at[didx.at[bk, g % SL]],
                    ssem).wait()

            slab_wait(0)
            offset_row(0)
            offset_row(1)
            gfire(0)
            gfire(1)

            def edge(g, _, rel=rel):
                gwait(g)
                sfire(g)
                gp = jnp.maximum(g - 1, 0)

                @pl.when(g >= 1)
                def _():
                    swait(gp)

                gn = jnp.minimum(g + 2, ROWS - 1)

                @pl.when(g + 2 <= ROWS - 1)
                def _():
                    kn = gn // SL

                    @pl.when(gn % SL == 0)
                    def _():
                        slab_wait(kn)

                    @pl.when(jnp.logical_and(gn % SL == 4,
                                             kn + 1 <= ROWS // SL - 1))
                    def _():
                        slab_fire(kn + 1)

                    offset_row(gn)
                    gfire(gn)

                return _

            lax.fori_loop(0, ROWS, edge, None)
            swait(ROWS - 1)
            plsc.subcore_barrier()

            padbase = (N + 8 + s * 8 - ZROWS) * nch + c_abs

            def flush(f, _, rel=rel):
                for b in range(2):
                    r0 = (2 * f + b) * ZROWS
                    base = (s * ZPT + r0) * nch + c_abs

                    def mkidx(k, _):
                        sl = pl.ds(k * 16, 16)
                        io = ionch[sl]
                        ibuf[b, sl] = io + jnp.where(
                            io <= (ZROWS - 1) * nch, base, padbase)
                        return _

                    lax.fori_loop(0, FPAD // 16, mkidx, None)
                    pltpu.sync_copy(acc.at[pl.ds(s * ZPT + r0, ZROWS)],
                                    fbuf.at[b, pl.ds(0, ZROWS)])
                d0 = pltpu.async_copy(fbuf.at[0], outs[rel].at[ibuf.at[0]],
                                      fs0)
                d1 = pltpu.async_copy(fbuf.at[1], outs[rel].at[ibuf.at[1]],
                                      fs1)
                d0.wait()
                d1.wait()
                return _

            lax.fori_loop(0, ZPT // ZROWS // 2, flush, None)
            plsc.subcore_barrier()
            return _

        lax.fori_loop(0, cps, chunk, None)


def _sc_scatter(nch, tabs, edges):
    out_t = tuple(jax.ShapeDtypeStruct((nch * ACC_ROWS, CW), jnp.float32)
                  for _ in range(4))
    k = pl.kernel(
        functools.partial(_scatter_body, nch),
        out_type=out_t,
        mesh=plsc.VectorSubcoreMesh(**_MESH),
        scratch_types=[
            pltpu.VMEM((2, SL, EB), jnp.int32),
            pltpu.VMEM((2, SL, EB), jnp.int32),
            pltpu.VMEM((3, EB, CW), jnp.float32),
            pltpu.VMEM((2, FPAD, CW), jnp.float32),
            pltpu.VMEM((2, FPAD), jnp.int32),
            pltpu.VMEM((FPAD,), jnp.int32),
            pltpu.VMEM_SHARED((ACC_ROWS, CW), jnp.float32),
            pltpu.SemaphoreType.DMA,
            pltpu.SemaphoreType.DMA,
            pltpu.SemaphoreType.DMA,
            pltpu.SemaphoreType.DMA,
            pltpu.SemaphoreType.DMA,
        ],
        compiler_params=pltpu.CompilerParams(use_tc_tiling_on_sc=False),
    )
    return k(*tabs, *edges)


def _mm1_body(x_ref, wg_ref, wl_ref, wr_ref, hist_ref, g_ref, p_ref, r_ref):
    x = x_ref[...]
    dinv = lax.rsqrt(hist_ref[...] + 1.0)
    g_ref[...] = jnp.dot(x, wg_ref[...],
                         preferred_element_type=jnp.float32) * dinv
    p_ref[...] = jnp.dot(x, wl_ref[...], preferred_element_type=jnp.float32)
    r_ref[...] = jnp.dot(x, wr_ref[...], preferred_element_type=jnp.float32)


def _mm1(x, wg, wl, wr, hist):
    blk = pl.BlockSpec((R, D_H), lambda i: (i, 0))
    return pl.pallas_call(
        _mm1_body,
        grid=(GRID,),
        in_specs=[
            pl.BlockSpec((R, D_IN), lambda i: (i, 0)),
            pl.BlockSpec((D_IN, D_H), lambda i: (0, 0)),
            pl.BlockSpec((D_IN, D_H), lambda i: (0, 0)),
            pl.BlockSpec((D_IN, D_H), lambda i: (0, 0)),
            pl.BlockSpec((R, 1), lambda i: (i, 0)),
        ],
        out_specs=[blk, blk, blk],
        out_shape=[jax.ShapeDtypeStruct((N, D_H), jnp.float32)] * 3,
    )(x, wg, wl, wr, hist)


def _mm2_body(s_ref, g_ref, a_ref, rr_ref, hist_ref, cnt_ref, b_ref,
              wg_ref, wl_ref, wr_ref, g2_ref, p2_ref, r2_ref):
    dinv = lax.rsqrt(hist_ref[...] + 1.0)
    cinv = 1.0 / jnp.maximum(cnt_ref[...], 1.0)
    t1 = jnp.maximum(dinv * (s_ref[...] + g_ref[...]) + a_ref[...] * cinv
                     + rr_ref[...] + b_ref[...], 0.0)
    h2 = jnp.dot(t1, wg_ref[...], preferred_element_type=jnp.float32)
    g2_ref[...] = h2 * dinv
    p2_ref[...] = jnp.dot(t1, wl_ref[...], preferred_element_type=jnp.float32)
    r2_ref[...] = jnp.dot(t1, wr_ref[...], preferred_element_type=jnp.float32)


def _mm2(s, g, a, rr, hist, cnt, b, wg, wl, wr):
    big = pl.BlockSpec((R, D_H), lambda i: (i, 0))
    col = pl.BlockSpec((R, 1), lambda i: (i, 0))
    w = pl.BlockSpec((D_H, D_OUT), lambda i: (0, 0))
    out = pl.BlockSpec((R, D_OUT), lambda i: (i, 0))
    return pl.pallas_call(
        _mm2_body,
        grid=(GRID,),
        in_specs=[big, big, big, big, col, col,
                  pl.BlockSpec((1, D_H), lambda i: (0, 0)), w, w, w],
        out_specs=[out, out, out],
        out_shape=[jax.ShapeDtypeStruct((N, D_OUT), jnp.float32)] * 3,
    )(s, g, a, rr, hist, cnt, b, wg, wl, wr)


def _fin_body(s_ref, g_ref, a_ref, r2_ref, hist_ref, cnt_ref, b_ref, out_ref):
    dinv = lax.rsqrt(hist_ref[...] + 1.0)
    cinv = 1.0 / jnp.maximum(cnt_ref[...], 1.0)
    out_ref[...] = (dinv * (s_ref[...] + g_ref[...]) + a_ref[...] * cinv
                    + b_ref[...] + r2_ref[...])


def _fin(s, g, a, r2, hist, cnt, b):
    sm = pl.BlockSpec((R, D_OUT), lambda i: (i, 0))
    col = pl.BlockSpec((R, 1), lambda i: (i, 0))
    return pl.pallas_call(
        _fin_body,
        grid=(GRID,),
        in_specs=[sm, sm, sm, sm, col, col,
                  pl.BlockSpec((1, D_OUT), lambda i: (0, 0))],
        out_specs=sm,
        out_shape=jax.ShapeDtypeStruct((N, D_OUT), jnp.float32),
    )(s, g, a, r2, hist, cnt, b)


def _prep_edges(ei):
    pad = E_PAD - E
    src = jnp.concatenate([ei[0], jnp.zeros((pad,), jnp.int32)])
    dst = jnp.concatenate([ei[1], jnp.full((pad,), DUMP, jnp.int32)])
    return src.reshape(NS, ROWS, EB), dst.reshape(NS, ROWS, EB)


def kernel(x_tx, x_addr, ei_tx_tx, ei_addr_addr, ei_addr_tx, ei_tx_addr,
           l1_tt_W, l1_tt_b, l1_aa_W, l1_aa_b,
           l1_at_Wl, l1_at_bl, l1_at_Wr,
           l1_ta_Wl, l1_ta_bl, l1_ta_Wr,
           l2_tt_W, l2_tt_b, l2_aa_W, l2_aa_b,
           l2_at_Wl, l2_at_bl, l2_at_Wr,
           l2_ta_Wl, l2_ta_bl, l2_ta_Wr):
    s_tt, d_tt = _prep_edges(ei_tx_tx)
    s_aa, d_aa = _prep_edges(ei_addr_addr)
    s_at, d_at = _prep_edges(ei_addr_tx)
    s_ta, d_ta = _prep_edges(ei_tx_addr)
    edges = (s_tt, d_tt, s_at, d_at, s_aa, d_aa, s_ta, d_ta)

    hist = _hist_kernel(d_tt, d_aa, d_at, d_ta).reshape(4, ACC_ROWS)
    h_tt = hist[0, :N].reshape(N, 1)
    h_aa = hist[1, :N].reshape(N, 1)
    c_at = hist[2, :N].reshape(N, 1)
    c_ta = hist[3, :N].reshape(N, 1)

    g_t, p_t, r_t = _mm1(x_tx, l1_tt_W, l1_ta_Wl, l1_at_Wr, h_tt)
    g_a, p_a, r_a = _mm1(x_addr, l1_aa_W, l1_at_Wl, l1_ta_Wr, h_aa)

    tabs1 = (g_t.reshape(C1 * N, CW), p_a.reshape(C1 * N, CW),
             g_a.reshape(C1 * N, CW), p_t.reshape(C1 * N, CW))
    S_tt, A_at, S_aa, A_ta = _sc_scatter(C1, tabs1, edges)

    b1_t = (l1_tt_b + l1_at_bl).reshape(1, D_H)
    b1_a = (l1_aa_b + l1_ta_bl).reshape(1, D_H)
    g2_t, p2_t, r2_t = _mm2(S_tt.reshape(ACC_ROWS, D_H), g_t,
                            A_at.reshape(ACC_ROWS, D_H), r_t, h_tt, c_at,
                            b1_t, l2_tt_W, l2_ta_Wl, l2_at_Wr)
    g2_a, p2_a, r2_a = _mm2(S_aa.reshape(ACC_ROWS, D_H), g_a,
                            A_ta.reshape(ACC_ROWS, D_H), r_a, h_aa, c_ta,
                            b1_a, l2_aa_W, l2_at_Wl, l2_ta_Wr)

    tabs2 = (g2_t.reshape(C2 * N, CW), p2_a.reshape(C2 * N, CW),
             g2_a.reshape(C2 * N, CW), p2_t.reshape(C2 * N, CW))
    S2_tt, A2_at, S2_aa, A2_ta = _sc_scatter(C2, tabs2, edges)

    b2_t = (l2_tt_b + l2_at_bl).reshape(1, D_OUT)
    b2_a = (l2_aa_b + l2_ta_bl).reshape(1, D_OUT)
    t2 = _fin(S2_tt.reshape(ACC_ROWS, D_OUT), g2_t,
              A2_at.reshape(ACC_ROWS, D_OUT), r2_t, h_tt, c_at, b2_t)
    a2 = _fin(S2_aa.reshape(ACC_ROWS, D_OUT), g2_a,
              A2_ta.reshape(ACC_ROWS, D_OUT), r2_a, h_aa, c_ta, b2_a)
    return (t2, a2)


# speedup vs baseline: 12.4958x; 1.0776x over previous
import functools

import jax
import jax.numpy as jnp
from jax import lax
from jax.experimental import pallas as pl
from jax.experimental.pallas import tpu as pltpu
from jax.experimental.pallas import tpu_sc as plsc

N = 50000
E = 400000
D_IN = 128
D_H = 128
D_OUT = 64

NC = 2
NS = 16
CW = 32

EB = 128
ROWS = 196
EPT_PAD = ROWS * EB
E_PAD = NS * EPT_PAD
DUMP = N
ACC_ROWS = 50176
ZPT = ACC_ROWS // NS
ZROWS = 28
FPAD = 32
NZ = 7
NB = 7
SL = 28

C1 = D_H // CW
C2 = D_OUT // CW
R = 1000
GRID = N // R

_MESH = dict(core_axis_name="c", subcore_axis_name="s")


def _hist_body(d_tt, d_aa, d_at, d_ta, hist_out, ones_v, zbuf, fbuf, dbuf,
               acc, hsem):
    core = lax.axis_index("c")
    s = lax.axis_index("s")
    d_refs = (d_tt, d_aa, d_at, d_ta)

    def fill(i, _):
        ones_v[pl.ds(i * 16, 16)] = jnp.ones((16,), jnp.float32)
        zbuf[pl.ds(i * 16, 16)] = jnp.zeros((16,), jnp.float32)
        return _

    lax.fori_loop(0, ZPT // 16, fill, None)

    def do_rel(rel):
        pltpu.sync_copy(zbuf, acc.at[pl.ds(s * ZPT, ZPT)])
        plsc.subcore_barrier()
        pltpu.sync_copy(d_refs[rel].at[s], dbuf)

        def step(g, _):
            ds = [pltpu.async_copy(ones_v.at[pl.ds(0, EB)],
                                   acc.at[dbuf.at[g * NB + b]], hsem,
                                   add=True)
                  for b in range(NB)]
            for d in ds:
                d.wait()
            return _

        lax.fori_loop(0, ROWS // NB, step, None)
        plsc.subcore_barrier()
        pltpu.sync_copy(acc.at[pl.ds(s * ZPT, ZPT)], fbuf)
        pltpu.sync_copy(fbuf, hist_out.at[pl.ds(rel * ACC_ROWS + s * ZPT,
                                                ZPT)])
        plsc.subcore_barrier()

    for c_py in range(NC):
        @pl.when(core == c_py)
        def _():
            for rel in (c_py, c_py + 2):
                do_rel(rel)


def _hist_kernel(d_tt, d_aa, d_at, d_ta):
    k = pl.kernel(
        _hist_body,
        out_type=jax.ShapeDtypeStruct((4 * ACC_ROWS,), jnp.float32),
        mesh=plsc.VectorSubcoreMesh(**_MESH),
        scratch_types=[
            pltpu.VMEM((ZPT,), jnp.float32),
            pltpu.VMEM((ZPT,), jnp.float32),
            pltpu.VMEM((ZPT,), jnp.float32),
            pltpu.VMEM((ROWS, EB), jnp.int32),
            pltpu.VMEM_SHARED((ACC_ROWS,), jnp.float32),
            pltpu.SemaphoreType.DMA,
        ],
    )
    return k(d_tt, d_aa, d_at, d_ta)


def _scatter_body(nch, rc, spec, *args):
    nt = 1 + max(sp[0] for sp in spec)
    no = 1 + max(sp[1] for sp in spec)
    cps = rc // NC
    tabs_u = args[:nt]
    e = args[nt:nt + 8]
    outs_u = args[nt + 8:nt + 8 + no]
    (sidx, didx, gbuf, fbuf, ibuf, ionch, acc,
     gsem, ssem, isem, fs0, fs1) = args[nt + 8 + no:]
    core = lax.axis_index("c")
    s = lax.axis_index("s")
    tabs = tuple(tabs_u[sp[0]] for sp in spec)
    outs = tuple(outs_u[sp[1]] for sp in spec)
    coffs = tuple(sp[2] for sp in spec)
    srcs = (e[0], e[2], e[4], e[6])
    dsts = (e[1], e[3], e[5], e[7])

    def fill_iota(k, _):
        ionch[pl.ds(k * 16, 16)] = (jnp.arange(16, dtype=jnp.int32)
                                    + k * 16) * nch
        return _

    lax.fori_loop(0, FPAD // 16, fill_iota, None)

    for rel in range(4):
        def chunk(ci, _, rel=rel):
            c_abs = coffs[rel] + core * cps + ci

            def slab_fire(k):
                bk = k % 2
                return (pltpu.async_copy(
                            srcs[rel].at[s, pl.ds(k * SL, SL)],
                            sidx.at[bk], isem),
                        pltpu.async_copy(
                            dsts[rel].at[s, pl.ds(k * SL, SL)],
                            didx.at[bk], isem))

            def slab_wait(k):
                bk = k % 2
                pltpu.make_async_copy(srcs[rel].at[s, pl.ds(k * SL, SL)],
                                      sidx.at[bk], isem).wait()
                pltpu.make_async_copy(dsts[rel].at[s, pl.ds(k * SL, SL)],
                                      didx.at[bk], isem).wait()

            def offset_row(g):
                bk = (g // SL) % 2
                row = g % SL
                for k in range(EB // 16):
                    sl = pl.ds(k * 16, 16)
                    sidx[bk, row, sl] = sidx[bk, row, sl] * nch + c_abs

            def zf(i, _):
                fbuf[0, i % ZROWS, pl.ds((i // ZROWS) * 16, 16)] = (
                    jnp.zeros((16,), jnp.float32))
                return _

            lax.fori_loop(0, ZROWS * (CW // 16), zf, None)
            slab_fire(0)
            slab_fire(1)

            def zero(zg, _):
                zd = [pltpu.async_copy(
                          fbuf.at[0, pl.ds(0, ZROWS)],
                          acc.at[pl.ds(s * ZPT + (zg * NZ + z) * ZROWS,
                                       ZROWS)], gsem)
                      for z in range(NZ)]
                for d in zd:
                    d.wait()
                return _

            lax.fori_loop(0, ZPT // ZROWS // NZ, zero, None)
            plsc.subcore_barrier()

            def gfire(g):
                bk = (g // SL) % 2
                return pltpu.async_copy(
                    tabs[rel].at[sidx.at[bk, g % SL]], gbuf.at[g % 3], gsem)

            def gwait(g):
                bk = (g // SL) % 2
                pltpu.make_async_copy(
                    tabs[rel].at[sidx.at[bk, g % SL]], gbuf.at[g % 3],
                    gsem).wait()

            def sfire(g):
                bk = (g // SL) % 2
                return pltpu.async_copy(
                    gbuf.at[g % 3], acc.at[didx.at[bk, g % SL]], ssem,
                    add=True)

            def swait(g):
                bk = (g // SL) % 2
                pltpu.make_async_copy(
                    gbuf.at[g % 3], acc.at[didx.at[bk, g % SL]],
                    ssem).wait()

            slab_wait(0)
            offset_row(0)
            offset_row(1)
            gfire(0)
            gfire(1)

            def edge(g, _, rel=rel):
                gwait(g)
                sfire(g)
                gp = jnp.maximum(g - 1, 0)

                @pl.when(g >= 1)
                def _():
                    swait(gp)

                gn = jnp.minimum(g + 2, ROWS - 1)

                @pl.when(g + 2 <= ROWS - 1)
                def _():
                    kn = gn // SL

                    @pl.when(gn % SL == 0)
                    def _():
                        slab_wait(kn)

                    @pl.when(jnp.logical_and(gn % SL == 4,
                                             kn + 1 <= ROWS // SL - 1))
                    def _():
                        slab_fire(kn + 1)

                    offset_row(gn)
                    gfire(gn)

                return _

            lax.fori_loop(0, ROWS, edge, None)
            swait(ROWS - 1)
            plsc.subcore_barrier()

            padbase = (N + 8 + s * 8 - ZROWS) * nch + c_abs

            def flush(f, _, rel=rel):
                for b in range(2):
                    r0 = (2 * f + b) * ZROWS
                    base = (s * ZPT + r0) * nch + c_abs

                    def mkidx(k, _):
                        sl = pl.ds(k * 16, 16)
                        io = ionch[sl]
                        ibuf[b, sl] = io + jnp.where(
                            io <= (ZROWS - 1) * nch, base, padbase)
                        return _

                    lax.fori_loop(0, FPAD // 16, mkidx, None)
                    pltpu.sync_copy(acc.at[pl.ds(s * ZPT + r0, ZROWS)],
                                    fbuf.at[b, pl.ds(0, ZROWS)])
                d0 = pltpu.async_copy(fbuf.at[0], outs[rel].at[ibuf.at[0]],
                                      fs0)
                d1 = pltpu.async_copy(fbuf.at[1], outs[rel].at[ibuf.at[1]],
                                      fs1)
                d0.wait()
                d1.wait()
                return _

            lax.fori_loop(0, ZPT // ZROWS // 2, flush, None)
            plsc.subcore_barrier()
            return _

        lax.fori_loop(0, cps, chunk, None)


def _sc_scatter(nch, rc, spec, tabs, edges):
    no = 1 + max(sp[1] for sp in spec)
    out_t = tuple(jax.ShapeDtypeStruct((nch * ACC_ROWS, CW), jnp.float32)
                  for _ in range(no))
    k = pl.kernel(
        functools.partial(_scatter_body, nch, rc, spec),
        out_type=out_t,
        mesh=plsc.VectorSubcoreMesh(**_MESH),
        scratch_types=[
            pltpu.VMEM((2, SL, EB), jnp.int32),
            pltpu.VMEM((2, SL, EB), jnp.int32),
            pltpu.VMEM((3, EB, CW), jnp.float32),
            pltpu.VMEM((2, FPAD, CW), jnp.float32),
            pltpu.VMEM((2, FPAD), jnp.int32),
            pltpu.VMEM((FPAD,), jnp.int32),
            pltpu.VMEM_SHARED((ACC_ROWS, CW), jnp.float32),
            pltpu.SemaphoreType.DMA,
            pltpu.SemaphoreType.DMA,
            pltpu.SemaphoreType.DMA,
            pltpu.SemaphoreType.DMA,
            pltpu.SemaphoreType.DMA,
        ],
        compiler_params=pltpu.CompilerParams(use_tc_tiling_on_sc=False),
    )
    return k(*tabs, *edges)


def _mm1_body(x_ref, wg_ref, wl_ref, wr_ref, hist_ref, g_ref, p_ref, r_ref):
    x = x_ref[...]
    dinv = lax.rsqrt(hist_ref[...] + 1.0)
    g_ref[...] = jnp.dot(x, wg_ref[...],
                         preferred_element_type=jnp.float32) * dinv
    p_ref[...] = jnp.dot(x, wl_ref[...], preferred_element_type=jnp.float32)
    r_ref[...] = jnp.dot(x, wr_ref[...], preferred_element_type=jnp.float32)


def _mm1(x, wg, wl, wr, hist):
    blk = pl.BlockSpec((R, D_H), lambda i: (i, 0))
    return pl.pallas_call(
        _mm1_body,
        grid=(GRID,),
        in_specs=[
            pl.BlockSpec((R, D_IN), lambda i: (i, 0)),
            pl.BlockSpec((D_IN, D_H), lambda i: (0, 0)),
            pl.BlockSpec((D_IN, D_H), lambda i: (0, 0)),
            pl.BlockSpec((D_IN, D_H), lambda i: (0, 0)),
            pl.BlockSpec((R, 1), lambda i: (i, 0)),
        ],
        out_specs=[blk, blk, blk],
        out_shape=[jax.ShapeDtypeStruct((N, D_H), jnp.float32)] * 3,
    )(x, wg, wl, wr, hist)


def _mm2_body(s_ref, g_ref, a_ref, rr_ref, hist_ref, cnt_ref, b_ref,
              wg_ref, wl_ref, wr_ref, gp_ref, r2_ref):
    dinv = lax.rsqrt(hist_ref[...] + 1.0)
    cinv = 1.0 / jnp.maximum(cnt_ref[...], 1.0)
    t1 = jnp.maximum(dinv * (s_ref[...] + g_ref[...]) + a_ref[...] * cinv
                     + rr_ref[...] + b_ref[...], 0.0)
    h2 = jnp.dot(t1, wg_ref[...], preferred_element_type=jnp.float32)
    gp_ref[:, :D_OUT] = h2 * dinv
    gp_ref[:, D_OUT:] = jnp.dot(t1, wl_ref[...],
                                preferred_element_type=jnp.float32)
    r2_ref[...] = jnp.dot(t1, wr_ref[...], preferred_element_type=jnp.float32)


def _mm2(s, g, a, rr, hist, cnt, b, wg, wl, wr):
    big = pl.BlockSpec((R, D_H), lambda i: (i, 0))
    col = pl.BlockSpec((R, 1), lambda i: (i, 0))
    w = pl.BlockSpec((D_H, D_OUT), lambda i: (0, 0))
    out = pl.BlockSpec((R, D_OUT), lambda i: (i, 0))
    return pl.pallas_call(
        _mm2_body,
        grid=(GRID,),
        in_specs=[big, big, big, big, col, col,
                  pl.BlockSpec((1, D_H), lambda i: (0, 0)), w, w, w],
        out_specs=[big, out],
        out_shape=[jax.ShapeDtypeStruct((N, D_H), jnp.float32),
                   jax.ShapeDtypeStruct((N, D_OUT), jnp.float32)],
    )(s, g, a, rr, hist, cnt, b, wg, wl, wr)


def _fin_body(sa_ref, gp_ref, r2_ref, hist_ref, cnt_ref, b_ref, out_ref):
    dinv = lax.rsqrt(hist_ref[...] + 1.0)
    cinv = 1.0 / jnp.maximum(cnt_ref[...], 1.0)
    out_ref[...] = (dinv * (sa_ref[:, :D_OUT] + gp_ref[:, :D_OUT])
                    + sa_ref[:, D_OUT:] * cinv + b_ref[...] + r2_ref[...])


def _fin(sa, gp, r2, hist, cnt, b):
    big = pl.BlockSpec((R, D_H), lambda i: (i, 0))
    sm = pl.BlockSpec((R, D_OUT), lambda i: (i, 0))
    col = pl.BlockSpec((R, 1), lambda i: (i, 0))
    return pl.pallas_call(
        _fin_body,
        grid=(GRID,),
        in_specs=[big, big, sm, col, col,
                  pl.BlockSpec((1, D_OUT), lambda i: (0, 0))],
        out_specs=sm,
        out_shape=jax.ShapeDtypeStruct((N, D_OUT), jnp.float32),
    )(sa, gp, r2, hist, cnt, b)


def _prep_edges(ei):
    pad = E_PAD - E
    src = jnp.concatenate([ei[0], jnp.zeros((pad,), jnp.int32)])
    dst = jnp.concatenate([ei[1], jnp.full((pad,), DUMP, jnp.int32)])
    return src.reshape(NS, ROWS, EB), dst.reshape(NS, ROWS, EB)


def kernel(x_tx, x_addr, ei_tx_tx, ei_addr_addr, ei_addr_tx, ei_tx_addr,
           l1_tt_W, l1_tt_b, l1_aa_W, l1_aa_b,
           l1_at_Wl, l1_at_bl, l1_at_Wr,
           l1_ta_Wl, l1_ta_bl, l1_ta_Wr,
           l2_tt_W, l2_tt_b, l2_aa_W, l2_aa_b,
           l2_at_Wl, l2_at_bl, l2_at_Wr,
           l2_ta_Wl, l2_ta_bl, l2_ta_Wr):
    s_tt, d_tt = _prep_edges(ei_tx_tx)
    s_aa, d_aa = _prep_edges(ei_addr_addr)
    s_at, d_at = _prep_edges(ei_addr_tx)
    s_ta, d_ta = _prep_edges(ei_tx_addr)
    edges = (s_tt, d_tt, s_at, d_at, s_aa, d_aa, s_ta, d_ta)

    hist = _hist_kernel(d_tt, d_aa, d_at, d_ta).reshape(4, ACC_ROWS)
    h_tt = hist[0, :N].reshape(N, 1)
    h_aa = hist[1, :N].reshape(N, 1)
    c_at = hist[2, :N].reshape(N, 1)
    c_ta = hist[3, :N].reshape(N, 1)

    g_t, p_t, r_t = _mm1(x_tx, l1_tt_W, l1_ta_Wl, l1_at_Wr, h_tt)
    g_a, p_a, r_a = _mm1(x_addr, l1_aa_W, l1_at_Wl, l1_ta_Wr, h_aa)

    tabs1 = (g_t.reshape(C1 * N, CW), p_a.reshape(C1 * N, CW),
             g_a.reshape(C1 * N, CW), p_t.reshape(C1 * N, CW))
    spec1 = ((0, 0, 0), (1, 1, 0), (2, 2, 0), (3, 3, 0))
    S_tt, A_at, S_aa, A_ta = _sc_scatter(C1, C1, spec1, tabs1, edges)

    b1_t = (l1_tt_b + l1_at_bl).reshape(1, D_H)
    b1_a = (l1_aa_b + l1_ta_bl).reshape(1, D_H)
    gp_t, r2_t = _mm2(S_tt.reshape(ACC_ROWS, D_H), g_t,
                      A_at.reshape(ACC_ROWS, D_H), r_t, h_tt, c_at,
                      b1_t, l2_tt_W, l2_ta_Wl, l2_at_Wr)
    gp_a, r2_a = _mm2(S_aa.reshape(ACC_ROWS, D_H), g_a,
                      A_ta.reshape(ACC_ROWS, D_H), r_a, h_aa, c_ta,
                      b1_a, l2_aa_W, l2_at_Wl, l2_ta_Wr)

    nch2 = D_H // CW
    tabs2 = (gp_t.reshape(nch2 * N, CW), gp_a.reshape(nch2 * N, CW))
    spec2 = ((0, 0, 0), (1, 0, 2), (1, 1, 0), (0, 1, 2))
    SA_t, SA_a = _sc_scatter(nch2, 2, spec2, tabs2, edges)

    b2_t = (l2_tt_b + l2_at_bl).reshape(1, D_OUT)
    b2_a = (l2_aa_b + l2_ta_bl).reshape(1, D_OUT)
    t2 = _fin(SA_t.reshape(ACC_ROWS, D_H), gp_t, r2_t, h_tt, c_at, b2_t)
    a2 = _fin(SA_a.reshape(ACC_ROWS, D_H), gp_a, r2_a, h_aa, c_ta, b2_a)
    return (t2, a2)


# speedup vs baseline: 13.0391x; 1.0435x over previous
import functools

import jax
import jax.numpy as jnp
from jax import lax
from jax.experimental import pallas as pl
from jax.experimental.pallas import tpu as pltpu
from jax.experimental.pallas import tpu_sc as plsc

N = 50000
E = 400000
D_IN = 128
D_H = 128
D_OUT = 64

NC = 2
NS = 16
CW = 32

EB = 128
ROWS = 196
EPT_PAD = ROWS * EB
E_PAD = NS * EPT_PAD
DUMP = N
ACC_ROWS = 50176
ZPT = ACC_ROWS // NS
ZROWS = 28
FPAD = 32
NZ = 7
NB = 7
SL = 28

C1 = D_H // CW
C2 = D_OUT // CW
R = 1000
GRID = N // R

_MESH = dict(core_axis_name="c", subcore_axis_name="s")


def _hist_body(d_tt, d_aa, d_at, d_ta, hist_out, ones_v, zbuf, fbuf, dbuf,
               acc, hsem):
    core = lax.axis_index("c")
    s = lax.axis_index("s")
    d_refs = (d_tt, d_aa, d_at, d_ta)

    def fill(i, _):
        ones_v[pl.ds(i * 16, 16)] = jnp.ones((16,), jnp.float32)
        zbuf[pl.ds(i * 16, 16)] = jnp.zeros((16,), jnp.float32)
        return _

    lax.fori_loop(0, ZPT // 16, fill, None)

    def do_rel(rel):
        pltpu.sync_copy(zbuf, acc.at[pl.ds(s * ZPT, ZPT)])
        plsc.subcore_barrier()
        pltpu.sync_copy(d_refs[rel].at[s], dbuf)

        def step(g, _):
            ds = [pltpu.async_copy(ones_v.at[pl.ds(0, EB)],
                                   acc.at[dbuf.at[g * NB + b]], hsem,
                                   add=True)
                  for b in range(NB)]
            for d in ds:
                d.wait()
            return _

        lax.fori_loop(0, ROWS // NB, step, None)
        plsc.subcore_barrier()
        pltpu.sync_copy(acc.at[pl.ds(s * ZPT, ZPT)], fbuf)
        pltpu.sync_copy(fbuf, hist_out.at[pl.ds(rel * ACC_ROWS + s * ZPT,
                                                ZPT)])
        plsc.subcore_barrier()

    for c_py in range(NC):
        @pl.when(core == c_py)
        def _():
            for rel in (c_py, c_py + 2):
                do_rel(rel)


def _hist_kernel(d_tt, d_aa, d_at, d_ta):
    k = pl.kernel(
        _hist_body,
        out_type=jax.ShapeDtypeStruct((4 * ACC_ROWS,), jnp.float32),
        mesh=plsc.VectorSubcoreMesh(**_MESH),
        scratch_types=[
            pltpu.VMEM((ZPT,), jnp.float32),
            pltpu.VMEM((ZPT,), jnp.float32),
            pltpu.VMEM((ZPT,), jnp.float32),
            pltpu.VMEM((ROWS, EB), jnp.int32),
            pltpu.VMEM_SHARED((ACC_ROWS,), jnp.float32),
            pltpu.SemaphoreType.DMA,
        ],
    )
    return k(d_tt, d_aa, d_at, d_ta)


def _scatter_body(nch, rc, spec, *args):
    nr = len(spec)
    nt = 1 + max(sp[0] for sp in spec)
    no = 1 + max(sp[1] for sp in spec)
    cps = rc // NC
    tabs_u = args[:nt]
    e = args[nt:nt + 2 * nr]
    outs_u = args[nt + 2 * nr:nt + 2 * nr + no]
    (sidx, didx, gbuf, fbuf, ibuf, ionch, acc,
     gsem, ssem, isem, fs0, fs1) = args[nt + 2 * nr + no:]
    core = lax.axis_index("c")
    s = lax.axis_index("s")
    tabs = tuple(tabs_u[sp[0]] for sp in spec)
    outs = tuple(outs_u[sp[1]] for sp in spec)
    coffs = tuple(sp[2] for sp in spec)
    srcs = e[0::2]
    dsts = e[1::2]

    def fill_iota(k, _):
        ionch[pl.ds(k * 16, 16)] = (jnp.arange(16, dtype=jnp.int32)
                                    + k * 16) * nch
        return _

    lax.fori_loop(0, FPAD // 16, fill_iota, None)

    for rel in range(len(spec)):
        def chunk(ci, _, rel=rel):
            c_abs = coffs[rel] + core * cps + ci

            def slab_fire(k):
                bk = k % 2
                return (pltpu.async_copy(
                            srcs[rel].at[s, pl.ds(k * SL, SL)],
                            sidx.at[bk], isem),
                        pltpu.async_copy(
                            dsts[rel].at[s, pl.ds(k * SL, SL)],
                            didx.at[bk], isem))

            def slab_wait(k):
                bk = k % 2
                pltpu.make_async_copy(srcs[rel].at[s, pl.ds(k * SL, SL)],
                                      sidx.at[bk], isem).wait()
                pltpu.make_async_copy(dsts[rel].at[s, pl.ds(k * SL, SL)],
                                      didx.at[bk], isem).wait()

            def offset_row(g):
                bk = (g // SL) % 2
                row = g % SL
                for k in range(EB // 16):
                    sl = pl.ds(k * 16, 16)
                    sidx[bk, row, sl] = sidx[bk, row, sl] * nch + c_abs

            def zf(i, _):
                fbuf[0, i % ZROWS, pl.ds((i // ZROWS) * 16, 16)] = (
                    jnp.zeros((16,), jnp.float32))
                return _

            lax.fori_loop(0, ZROWS * (CW // 16), zf, None)
            slab_fire(0)
            slab_fire(1)

            def zero(zg, _):
                zd = [pltpu.async_copy(
                          fbuf.at[0, pl.ds(0, ZROWS)],
                          acc.at[pl.ds(s * ZPT + (zg * NZ + z) * ZROWS,
                                       ZROWS)], gsem)
                      for z in range(NZ)]
                for d in zd:
                    d.wait()
                return _

            lax.fori_loop(0, ZPT // ZROWS // NZ, zero, None)
            plsc.subcore_barrier()

            def gfire(g):
                bk = (g // SL) % 2
                return pltpu.async_copy(
                    tabs[rel].at[sidx.at[bk, g % SL]], gbuf.at[g % 3], gsem)

            def gwait(g):
                bk = (g // SL) % 2
                pltpu.make_async_copy(
                    tabs[rel].at[sidx.at[bk, g % SL]], gbuf.at[g % 3],
                    gsem).wait()

            def sfire(g):
                bk = (g // SL) % 2
                return pltpu.async_copy(
                    gbuf.at[g % 3], acc.at[didx.at[bk, g % SL]], ssem,
                    add=True)

            def swait(g):
                bk = (g // SL) % 2
                pltpu.make_async_copy(
                    gbuf.at[g % 3], acc.at[didx.at[bk, g % SL]],
                    ssem).wait()

            slab_wait(0)
            offset_row(0)
            offset_row(1)
            gfire(0)
            gfire(1)

            def edge(g, _, rel=rel):
                gwait(g)
                sfire(g)
                gp = jnp.maximum(g - 1, 0)

                @pl.when(g >= 1)
                def _():
                    swait(gp)

                gn = jnp.minimum(g + 2, ROWS - 1)

                @pl.when(g + 2 <= ROWS - 1)
                def _():
                    kn = gn // SL

                    @pl.when(gn % SL == 0)
                    def _():
                        slab_wait(kn)

                    @pl.when(jnp.logical_and(gn % SL == 4,
                                             kn + 1 <= ROWS // SL - 1))
                    def _():
                        slab_fire(kn + 1)

                    offset_row(gn)
                    gfire(gn)

                return _

            lax.fori_loop(0, ROWS, edge, None)
            swait(ROWS - 1)
            plsc.subcore_barrier()

            padbase = (N + 8 + s * 8 - ZROWS) * nch + c_abs

            def flush(f, _, rel=rel):
                for b in range(2):
                    r0 = (2 * f + b) * ZROWS
                    base = (s * ZPT + r0) * nch + c_abs

                    def mkidx(k, _):
                        sl = pl.ds(k * 16, 16)
                        io = ionch[sl]
                        ibuf[b, sl] = io + jnp.where(
                            io <= (ZROWS - 1) * nch, base, padbase)
                        return _

                    lax.fori_loop(0, FPAD // 16, mkidx, None)
                    pltpu.sync_copy(acc.at[pl.ds(s * ZPT + r0, ZROWS)],
                                    fbuf.at[b, pl.ds(0, ZROWS)])
                d0 = pltpu.async_copy(fbuf.at[0], outs[rel].at[ibuf.at[0]],
                                      fs0)
                d1 = pltpu.async_copy(fbuf.at[1], outs[rel].at[ibuf.at[1]],
                                      fs1)
                d0.wait()
                d1.wait()
                return _

            lax.fori_loop(0, ZPT // ZROWS // 2, flush, None)
            plsc.subcore_barrier()
            return _

        lax.fori_loop(0, cps, chunk, None)


def _sc_scatter(nch, rc, spec, tabs, edges):
    no = 1 + max(sp[1] for sp in spec)
    out_t = tuple(jax.ShapeDtypeStruct((nch * ACC_ROWS, CW), jnp.float32)
                  for _ in range(no))
    k = pl.kernel(
        functools.partial(_scatter_body, nch, rc, spec),
        out_type=out_t,
        mesh=plsc.VectorSubcoreMesh(**_MESH),
        scratch_types=[
            pltpu.VMEM((2, SL, EB), jnp.int32),
            pltpu.VMEM((2, SL, EB), jnp.int32),
            pltpu.VMEM((3, EB, CW), jnp.float32),
            pltpu.VMEM((2, FPAD, CW), jnp.float32),
            pltpu.VMEM((2, FPAD), jnp.int32),
            pltpu.VMEM((FPAD,), jnp.int32),
            pltpu.VMEM_SHARED((ACC_ROWS, CW), jnp.float32),
            pltpu.SemaphoreType.DMA,
            pltpu.SemaphoreType.DMA,
            pltpu.SemaphoreType.DMA,
            pltpu.SemaphoreType.DMA,
            pltpu.SemaphoreType.DMA,
        ],
        compiler_params=pltpu.CompilerParams(use_tc_tiling_on_sc=False),
    )
    return k(*tabs, *edges)


def _mm1_body(x_ref, wg_ref, wl_ref, wr_ref, hist_ref, g_ref, p_ref, r_ref):
    x = x_ref[...]
    dinv = lax.rsqrt(hist_ref[...] + 1.0)
    g_ref[...] = jnp.dot(x, wg_ref[...],
                         preferred_element_type=jnp.float32) * dinv
    p_ref[...] = jnp.dot(x, wl_ref[...], preferred_element_type=jnp.float32)
    r_ref[...] = jnp.dot(x, wr_ref[...], preferred_element_type=jnp.float32)


def _mm1(x, wg, wl, wr, hist):
    blk = pl.BlockSpec((R, D_H), lambda i: (i, 0))
    return pl.pallas_call(
        _mm1_body,
        grid=(GRID,),
        in_specs=[
            pl.BlockSpec((R, D_IN), lambda i: (i, 0)),
            pl.BlockSpec((D_IN, D_H), lambda i: (0, 0)),
            pl.BlockSpec((D_IN, D_H), lambda i: (0, 0)),
            pl.BlockSpec((D_IN, D_H), lambda i: (0, 0)),
            pl.BlockSpec((R, 1), lambda i: (i, 0)),
        ],
        out_specs=[blk, blk, blk],
        out_shape=[jax.ShapeDtypeStruct((N, D_H), jnp.float32)] * 3,
    )(x, wg, wl, wr, hist)


def _mm2_body(s_ref, g_ref, a_ref, rr_ref, hist_ref, cnt_ref, b_ref,
              wg_ref, wl_ref, wr_ref, gp_ref, r2_ref):
    dinv = lax.rsqrt(hist_ref[...] + 1.0)
    cinv = 1.0 / jnp.maximum(cnt_ref[...], 1.0)
    t1 = jnp.maximum(dinv * (s_ref[...] + g_ref[...]) + a_ref[...] * cinv
                     + rr_ref[...] + b_ref[...], 0.0)
    h2 = jnp.dot(t1, wg_ref[...], preferred_element_type=jnp.float32)
    gp_ref[:, :D_OUT] = h2 * dinv
    gp_ref[:, D_OUT:] = jnp.dot(t1, wl_ref[...],
                                preferred_element_type=jnp.float32)
    r2_ref[...] = jnp.dot(t1, wr_ref[...], preferred_element_type=jnp.float32)


def _mm2(s, g, a, rr, hist, cnt, b, wg, wl, wr):
    big = pl.BlockSpec((R, D_H), lambda i: (i, 0))
    col = pl.BlockSpec((R, 1), lambda i: (i, 0))
    w = pl.BlockSpec((D_H, D_OUT), lambda i: (0, 0))
    out = pl.BlockSpec((R, D_OUT), lambda i: (i, 0))
    return pl.pallas_call(
        _mm2_body,
        grid=(GRID,),
        in_specs=[big, big, big, big, col, col,
                  pl.BlockSpec((1, D_H), lambda i: (0, 0)), w, w, w],
        out_specs=[big, out],
        out_shape=[jax.ShapeDtypeStruct((N, D_H), jnp.float32),
                   jax.ShapeDtypeStruct((N, D_OUT), jnp.float32)],
    )(s, g, a, rr, hist, cnt, b, wg, wl, wr)


def _fin_body(sa_ref, gp_ref, r2_ref, hist_ref, cnt_ref, b_ref, out_ref):
    dinv = lax.rsqrt(hist_ref[...] + 1.0)
    cinv = 1.0 / jnp.maximum(cnt_ref[...], 1.0)
    out_ref[...] = (dinv * (sa_ref[:, :D_OUT] + gp_ref[:, :D_OUT])
                    + sa_ref[:, D_OUT:] * cinv + b_ref[...] + r2_ref[...])


def _fin(sa, gp, r2, hist, cnt, b):
    big = pl.BlockSpec((R, D_H), lambda i: (i, 0))
    sm = pl.BlockSpec((R, D_OUT), lambda i: (i, 0))
    col = pl.BlockSpec((R, 1), lambda i: (i, 0))
    return pl.pallas_call(
        _fin_body,
        grid=(GRID,),
        in_specs=[big, big, sm, col, col,
                  pl.BlockSpec((1, D_OUT), lambda i: (0, 0))],
        out_specs=sm,
        out_shape=jax.ShapeDtypeStruct((N, D_OUT), jnp.float32),
    )(sa, gp, r2, hist, cnt, b)


def _prep_edges(ei):
    pad = E_PAD - E
    src = jnp.concatenate([ei[0], jnp.zeros((pad,), jnp.int32)])
    dst = jnp.concatenate([ei[1], jnp.full((pad,), DUMP, jnp.int32)])
    return src.reshape(NS, ROWS, EB), dst.reshape(NS, ROWS, EB)


def kernel(x_tx, x_addr, ei_tx_tx, ei_addr_addr, ei_addr_tx, ei_tx_addr,
           l1_tt_W, l1_tt_b, l1_aa_W, l1_aa_b,
           l1_at_Wl, l1_at_bl, l1_at_Wr,
           l1_ta_Wl, l1_ta_bl, l1_ta_Wr,
           l2_tt_W, l2_tt_b, l2_aa_W, l2_aa_b,
           l2_at_Wl, l2_at_bl, l2_at_Wr,
           l2_ta_Wl, l2_ta_bl, l2_ta_Wr):
    s_tt, d_tt = _prep_edges(ei_tx_tx)
    s_aa, d_aa = _prep_edges(ei_addr_addr)
    s_at, d_at = _prep_edges(ei_addr_tx)
    s_ta, d_ta = _prep_edges(ei_tx_addr)
    edges = (s_tt, d_tt, s_at, d_at, s_aa, d_aa, s_ta, d_ta)

    hist = _hist_kernel(d_tt, d_aa, d_at, d_ta).reshape(4, ACC_ROWS)
    h_tt = hist[0, :N].reshape(N, 1)
    h_aa = hist[1, :N].reshape(N, 1)
    c_at = hist[2, :N].reshape(N, 1)
    c_ta = hist[3, :N].reshape(N, 1)

    g_t, p_t, r_t = _mm1(x_tx, l1_tt_W, l1_ta_Wl, l1_at_Wr, h_tt)
    g_a, p_a, r_a = _mm1(x_addr, l1_aa_W, l1_at_Wl, l1_ta_Wr, h_aa)

    spec_l1 = ((0, 0, 0), (1, 1, 0))
    S_tt, A_at = _sc_scatter(
        C1, C1, spec_l1,
        (g_t.reshape(C1 * N, CW), p_a.reshape(C1 * N, CW)),
        (s_tt, d_tt, s_at, d_at))
    S_aa, A_ta = _sc_scatter(
        C1, C1, spec_l1,
        (g_a.reshape(C1 * N, CW), p_t.reshape(C1 * N, CW)),
        (s_aa, d_aa, s_ta, d_ta))

    b1_t = (l1_tt_b + l1_at_bl).reshape(1, D_H)
    b1_a = (l1_aa_b + l1_ta_bl).reshape(1, D_H)
    gp_t, r2_t = _mm2(S_tt.reshape(ACC_ROWS, D_H), g_t,
                      A_at.reshape(ACC_ROWS, D_H), r_t, h_tt, c_at,
                      b1_t, l2_tt_W, l2_ta_Wl, l2_at_Wr)
    gp_a, r2_a = _mm2(S_aa.reshape(ACC_ROWS, D_H), g_a,
                      A_ta.reshape(ACC_ROWS, D_H), r_a, h_aa, c_ta,
                      b1_a, l2_aa_W, l2_at_Wl, l2_ta_Wr)

    nch2 = D_H // CW
    gpt = gp_t.reshape(nch2 * N, CW)
    gpa = gp_a.reshape(nch2 * N, CW)
    spec_l2 = ((0, 0, 0), (1, 0, 2))
    (SA_t,) = _sc_scatter(nch2, 2, spec_l2, (gpt, gpa),
                          (s_tt, d_tt, s_at, d_at))
    (SA_a,) = _sc_scatter(nch2, 2, spec_l2, (gpa, gpt),
                          (s_aa, d_aa, s_ta, d_ta))

    b2_t = (l2_tt_b + l2_at_bl).reshape(1, D_OUT)
    b2_a = (l2_aa_b + l2_ta_bl).reshape(1, D_OUT)
    t2 = _fin(SA_t.reshape(ACC_ROWS, D_H), gp_t, r2_t, h_tt, c_at, b2_t)
    a2 = _fin(SA_a.reshape(ACC_ROWS, D_H), gp_a, r2_a, h_aa, c_ta, b2_a)
    return (t2, a2)
